# baseline (speedup 1.0000x reference)
import functools
import math

import jax
import jax.numpy as jnp
from jax.experimental import pallas as pl
from jax.experimental.pallas import tpu as pltpu

_EPS = 1e-5
_VMEM = 64 * 1024 * 1024


def _lstm_cell(z, c_prev, hp):
    i = jax.nn.sigmoid(z[:, 0 * hp:1 * hp])
    f = jax.nn.sigmoid(z[:, 1 * hp:2 * hp])
    g = jnp.tanh(z[:, 2 * hp:3 * hp])
    o = jax.nn.sigmoid(z[:, 3 * hp:4 * hp])
    c = f * c_prev + i * g
    return o * jnp.tanh(c), c


def _feat_kernel(*refs, T, B, dims):
    (xt, w1t, b1t, whh1t, w2t, b2t, lngt, lnbt, whh2t,
     xv, w1v, b1v, whh1v, w2v, b2v, lngv, lnbv, whh2v,
     xa, w1a, b1a, whh1a, w2a, b2a, lnga, lnba, whh2a,
     ut_ref, uv_ref, ua_ref,
     gt, gv, ga, h1t, h1v, h1a,
     hst, cst, hsv, csv, hsa, csa,
     w2tb, w2vp, w2ap) = refs

    mods = []
    for (H, Hp), g_scr, h1, hs, cs, whh1, whh2, x, w1, b1, w2, b2, lng, \
            lnb, out in (
            (dims[0], gt, h1t, hst, cst, whh1t, whh2t, xt, w1t, b1t, w2t,
             b2t, lngt, lnbt, ut_ref),
            (dims[1], gv, h1v, hsv, csv, whh1v, whh2v, xv, w1v, b1v, w2v,
             b2v, lngv, lnbv, uv_ref),
            (dims[2], ga, h1a, hsa, csa, whh1a, whh2a, xa, w1a, b1a, w2a,
             b2a, lnga, lnba, ua_ref)):
        mods.append(dict(H=H, Hp=Hp, g=g_scr, h1=h1, hs=hs, cs=cs,
                         whh1=whh1, whh2=whh2, x=x, w1=w1, b1=b1, w2=w2,
                         b2=b2, lng=lng, lnb=lnb, out=out))

    w2scrs = (w2tb, w2vp, w2ap)
    for scr, m in zip(w2scrs, mods):
        H, Hp = m["H"], m["Hp"]
        N = scr.shape[1]
        if 2 * H >= Hp:
            scr[0:H, :] = m["w2"][H:2 * H, :]
            scr[H:Hp, :] = jnp.zeros((Hp - H, N), jnp.float32)
        else:
            scr[0:H, :] = m["w2"][0:H, :]
            scr[H:Hp, :] = jnp.zeros((Hp - H, N), jnp.float32)
            scr[Hp:Hp + H, :] = m["w2"][H:2 * H, :]
            scr[Hp + H:2 * Hp, :] = jnp.zeros((Hp - H, N), jnp.float32)

    for m in mods:
        m["g"][...] = (
            jnp.dot(m["x"][...], m["w1"][...],
                    preferred_element_type=jnp.float32) + m["b1"][...]
        )

    def run_layer(whh_key, store_h):
        for m in mods:
            W2 = 2 * m["Hp"]
            m["hs"][...] = jnp.zeros((B, W2), jnp.float32)
            m["cs"][...] = jnp.zeros((B, W2), jnp.float32)
        for t in range(T):
            zs = []
            for m in mods:
                Hp, G4 = m["Hp"], 4 * m["Hp"]
                hf16 = m["hs"][:, 0:Hp].astype(jnp.bfloat16)
                hb16 = m["hs"][:, Hp:2 * Hp].astype(jnp.bfloat16)
                zf = m["g"][t * B:(t + 1) * B, 0:G4] + jnp.dot(
                    hf16, m[whh_key][0],
                    preferred_element_type=jnp.float32)
                zb = m["g"][(T - 1 - t) * B:(T - t) * B, G4:2 * G4] + \
                    jnp.dot(hb16, m[whh_key][1],
                            preferred_element_type=jnp.float32)
                zs.append((zf, zb))
            for m, (zf, zb) in zip(mods, zs):
                Hp = m["Hp"]
                hf, cf = _lstm_cell(zf, m["cs"][:, 0:Hp], Hp)
                hb, cb = _lstm_cell(zb, m["cs"][:, Hp:2 * Hp], Hp)
                m["hs"][:, 0:Hp] = hf
                m["hs"][:, Hp:2 * Hp] = hb
                m["cs"][:, 0:Hp] = cf
                m["cs"][:, Hp:2 * Hp] = cb
                if store_h:
                    m["h1"][t * B:(t + 1) * B, 0:Hp] = hf
                    m["h1"][(T - 1 - t) * B:(T - t) * B, Hp:2 * Hp] = hb

    run_layer("whh1", store_h=True)
    for m in mods:
        Hp = m["Hp"]
        m["out"][:, 0 * Hp:1 * Hp] = m["hs"][:, 0:Hp]
        m["out"][:, 2 * Hp:3 * Hp] = m["hs"][:, Hp:2 * Hp]

    for scr, m in zip(w2scrs, mods):
        H, Hp = m["H"], m["Hp"]
        x1 = m["h1"][...]
        inv_n = 1.0 / (2 * H)
        mu = jnp.sum(x1, axis=1, keepdims=True) * inv_n
        ex2 = jnp.sum(x1 * x1, axis=1, keepdims=True) * inv_n
        xn = (x1 - mu) * jax.lax.rsqrt(ex2 - mu * mu + _EPS) * \
            m["lng"][...] + m["lnb"][...]
        if 2 * H >= Hp:
            m["g"][...] = (
                jnp.dot(xn[:, 0:Hp], m["w2"][0:Hp, :],
                        preferred_element_type=jnp.float32)
                + jnp.dot(xn[:, Hp:2 * Hp], scr[...],
                          preferred_element_type=jnp.float32)
                + m["b2"][...]
            )
        else:
            m["g"][...] = (
                jnp.dot(xn, scr[...], preferred_element_type=jnp.float32)
                + m["b2"][...]
            )

    run_layer("whh2", store_h=False)
    for m in mods:
        Hp = m["Hp"]
        m["out"][:, 1 * Hp:2 * Hp] = m["hs"][:, 0:Hp]
        m["out"][:, 3 * Hp:4 * Hp] = m["hs"][:, Hp:2 * Hp]


def _head_kernel(*refs, nhead, dims):
    (ut, uv, ua,
     pwt, pbt, pgt, ptt, pwv, pbv, pgv, ptv, pwa, pba, pga, pta,
     qtw, qtb, qvw, qvb, qaw, qab, shw, shb, sdw, sdb,
     inw, inb, ouw, oub, l1g, l1b, f1w, f1b, f2w, f2b, l2g, l2b,
     fw, fb, o_ref, st_ref, sv_ref, sa_ref, ss_ref,
     pts, pvs, pas) = refs

    E = shw.shape[0]
    B = ut.shape[0]
    S = 6
    SB = S * B
    dh = E // nhead
    scale = 1.0 / math.sqrt(dh)

    for scr, w, (H, Hp) in ((pts, pwt, dims[0]), (pvs, pwv, dims[1]),
                            (pas, pwa, dims[2])):
        if H == Hp:
            scr[...] = w[...]
        else:
            for k in range(4):
                scr[k * Hp:k * Hp + H, :] = w[k * H:(k + 1) * H, :]
                scr[k * Hp + H:(k + 1) * Hp, :] = jnp.zeros(
                    (Hp - H, E), jnp.float32)
    pwt, pwv, pwa = pts, pvs, pas

    def ln(x, g, b):
        mu = jnp.mean(x, axis=-1, keepdims=True)
        xc = x - mu
        var = jnp.mean(xc * xc, axis=-1, keepdims=True)
        return xc * jax.lax.rsqrt(var + _EPS) * g[...] + b[...]

    def lin(x, w, b):
        return jnp.dot(x, w[...], preferred_element_type=jnp.float32) + b[...]

    t = ln(jnp.maximum(lin(ut[...], pwt, pbt), 0.0), pgt, ptt)
    v = ln(jnp.maximum(lin(uv[...], pwv, pbv), 0.0), pgv, ptv)
    a = ln(jnp.maximum(lin(ua[...], pwa, pba), 0.0), pga, pta)

    p_t = jax.nn.sigmoid(lin(t, qtw, qtb))
    p_v = jax.nn.sigmoid(lin(v, qvw, qvb))
    p_a = jax.nn.sigmoid(lin(a, qaw, qab))
    s_t = jax.nn.sigmoid(lin(t, shw, shb))
    s_v = jax.nn.sigmoid(lin(v, shw, shb))
    s_a = jax.nn.sigmoid(lin(a, shw, shb))

    st_ref[...] = lin(p_t, sdw, sdb)
    sv_ref[...] = lin(p_v, sdw, sdb)
    sa_ref[...] = lin(p_a, sdw, sdb)
    ss_ref[...] = lin((s_t + s_v + s_a) / 3.0, sdw, sdb)

    h = jnp.concatenate([p_t, p_v, p_a, s_t, s_v, s_a], axis=0)

    qkv = lin(h, inw, inb)
    q, k, vv = qkv[:, :E], qkv[:, E:2 * E], qkv[:, 2 * E:]
    ri = jax.lax.broadcasted_iota(jnp.int32, (SB, SB), 0)
    rj = jax.lax.broadcasted_iota(jnp.int32, (SB, SB), 1)
    same = (ri % B) == (rj % B)

    attn = jnp.zeros((SB, E), jnp.float32)
    for hd in range(nhead):
        cs = slice(hd * dh, (hd + 1) * dh)
        sc = jax.lax.dot_general(
            q[:, cs], k[:, cs], dimension_numbers=(((1,), (1,)), ((), ())),
            preferred_element_type=jnp.float32) * scale
        sc = jnp.where(same, sc, -1e30)
        m = jnp.max(sc, axis=-1, keepdims=True)
        p = jnp.exp(sc - m)
        p = p / jnp.sum(p, axis=-1, keepdims=True)
        hv = jnp.dot(p, vv[:, cs], preferred_element_type=jnp.float32)
        attn = attn + jnp.dot(hv, ouw[cs, :],
                              preferred_element_type=jnp.float32)

    x = ln(h + attn + oub[...], l1g, l1b)
    x = ln(x + lin(jnp.maximum(lin(x, f1w, f1b), 0.0), f2w, f2b), l2g, l2b)

    o = jnp.zeros((B, fw.shape[1]), jnp.float32)
    for s in range(S):
        o = o + jnp.dot(x[s * B:(s + 1) * B, :], fw[s * E:(s + 1) * E, :],
                        preferred_element_type=jnp.float32)
    o_ref[...] = o + fb[...]


def _pad_vec(g, H, Hp):
    if H == Hp:
        return g.reshape(1, -1)
    z = jnp.zeros((Hp - H,), g.dtype)
    return jnp.concatenate([g[:H], z, g[H:], z]).reshape(1, -1)


def kernel(trnn1_w_ih, trnn1_b, trnn1_w_hh, trnn2_w_ih, trnn2_b, trnn2_w_hh,
           vrnn1_w_ih, vrnn1_b, vrnn1_w_hh, vrnn2_w_ih, vrnn2_b, vrnn2_w_hh,
           arnn1_w_ih, arnn1_b, arnn1_w_hh, arnn2_w_ih, arnn2_b, arnn2_w_hh,
           tln_g, tln_b, vln_g, vln_b, aln_g, aln_b,
           proj_t_w, proj_t_b, proj_t_ln_g, proj_t_ln_b,
           proj_v_w, proj_v_b, proj_v_ln_g, proj_v_ln_b,
           proj_a_w, proj_a_b, proj_a_ln_g, proj_a_ln_b,
           priv_t_w, priv_t_b, priv_v_w, priv_v_b, priv_a_w, priv_a_b,
           shared_w, shared_b, spd_w, spd_b, fusion_w, fusion_b,
           tx_in_w, tx_in_b, tx_out_w, tx_out_b,
           tx_ff1_w, tx_ff1_b, tx_ff2_w, tx_ff2_b,
           tx_ln1_g, tx_ln1_b, tx_ln2_g, tx_ln2_b,
           visual, acoustic, sentences):
    B, T, Ht = sentences.shape
    Hv = visual.shape[2]
    Ha = acoustic.shape[2]
    Hpt = trnn1_w_hh.shape[1]
    Hpv = vrnn1_w_hh.shape[1]
    Hpa = arnn1_w_hh.shape[1]

    xt = jnp.transpose(sentences, (1, 0, 2)).reshape(T * B, Ht)
    xv = jnp.transpose(visual, (1, 0, 2)).reshape(T * B, Hv)
    xa = jnp.transpose(acoustic, (1, 0, 2)).reshape(T * B, Ha)

    r = lambda z: z.reshape(1, -1)
    bf = lambda z: z.astype(jnp.bfloat16)
    feat_in = (
        xt, trnn1_w_ih, r(trnn1_b), bf(trnn1_w_hh),
        trnn2_w_ih, r(trnn2_b),
        _pad_vec(tln_g, Ht, Hpt), _pad_vec(tln_b, Ht, Hpt), bf(trnn2_w_hh),
        xv, vrnn1_w_ih, r(vrnn1_b), bf(vrnn1_w_hh),
        vrnn2_w_ih, r(vrnn2_b),
        _pad_vec(vln_g, Hv, Hpv), _pad_vec(vln_b, Hv, Hpv), bf(vrnn2_w_hh),
        xa, arnn1_w_ih, r(arnn1_b), bf(arnn1_w_hh),
        arnn2_w_ih, r(arnn2_b),
        _pad_vec(aln_g, Ha, Hpa), _pad_vec(aln_b, Ha, Hpa), bf(arnn2_w_hh),
    )

    ut, uv, ua = pl.pallas_call(
        functools.partial(_feat_kernel, T=T, B=B,
                          dims=((Ht, Hpt), (Hv, Hpv), (Ha, Hpa))),
        out_shape=[
            jax.ShapeDtypeStruct((B, 4 * Hpt), jnp.float32),
            jax.ShapeDtypeStruct((B, 4 * Hpv), jnp.float32),
            jax.ShapeDtypeStruct((B, 4 * Hpa), jnp.float32),
        ],
        scratch_shapes=[
            pltpu.VMEM((T * B, 8 * Hpt), jnp.float32),
            pltpu.VMEM((T * B, 8 * Hpv), jnp.float32),
            pltpu.VMEM((T * B, 8 * Hpa), jnp.float32),
            pltpu.VMEM((T * B, 2 * Hpt), jnp.float32),
            pltpu.VMEM((T * B, 2 * Hpv), jnp.float32),
            pltpu.VMEM((T * B, 2 * Hpa), jnp.float32),
            pltpu.VMEM((B, 2 * Hpt), jnp.float32),
            pltpu.VMEM((B, 2 * Hpt), jnp.float32),
            pltpu.VMEM((B, 2 * Hpv), jnp.float32),
            pltpu.VMEM((B, 2 * Hpv), jnp.float32),
            pltpu.VMEM((B, 2 * Hpa), jnp.float32),
            pltpu.VMEM((B, 2 * Hpa), jnp.float32),
            pltpu.VMEM((Hpt if 2 * Ht >= Hpt else 2 * Hpt, 8 * Hpt),
                       jnp.float32),
            pltpu.VMEM((Hpv if 2 * Hv >= Hpv else 2 * Hpv, 8 * Hpv),
                       jnp.float32),
            pltpu.VMEM((Hpa if 2 * Ha >= Hpa else 2 * Hpa, 8 * Hpa),
                       jnp.float32),
        ],
        compiler_params=pltpu.CompilerParams(vmem_limit_bytes=_VMEM),
    )(*feat_in)

    head_in = (
        ut, uv, ua,
        proj_t_w, r(proj_t_b),
        r(proj_t_ln_g), r(proj_t_ln_b),
        proj_v_w, r(proj_v_b),
        r(proj_v_ln_g), r(proj_v_ln_b),
        proj_a_w, r(proj_a_b),
        r(proj_a_ln_g), r(proj_a_ln_b),
        priv_t_w, r(priv_t_b), priv_v_w, r(priv_v_b), priv_a_w, r(priv_a_b),
        shared_w, r(shared_b), spd_w, r(spd_b),
        tx_in_w, r(tx_in_b), tx_out_w, r(tx_out_b),
        r(tx_ln1_g), r(tx_ln1_b),
        tx_ff1_w, r(tx_ff1_b), tx_ff2_w, r(tx_ff2_b),
        r(tx_ln2_g), r(tx_ln2_b),
        fusion_w, r(fusion_b),
    )
    E = shared_w.shape[0]
    o, spt, spv, spa, sps = pl.pallas_call(
        functools.partial(_head_kernel, nhead=2,
                          dims=((Ht, Hpt), (Hv, Hpv), (Ha, Hpa))),
        out_shape=(
            jax.ShapeDtypeStruct((B, 3 * E), jnp.float32),
            jax.ShapeDtypeStruct((B, 4), jnp.float32),
            jax.ShapeDtypeStruct((B, 4), jnp.float32),
            jax.ShapeDtypeStruct((B, 4), jnp.float32),
            jax.ShapeDtypeStruct((B, 4), jnp.float32),
        ),
        scratch_shapes=[
            pltpu.VMEM((4 * Hpt, E), jnp.float32),
            pltpu.VMEM((4 * Hpv, E), jnp.float32),
            pltpu.VMEM((4 * Hpa, E), jnp.float32),
        ],
        compiler_params=pltpu.CompilerParams(vmem_limit_bytes=_VMEM),
    )(*head_in)
    aux = {"sp_p_t": spt, "sp_p_v": spv, "sp_p_a": spa, "sp_s": sps}
    return o, aux


# speedup vs baseline: 1.1030x; 1.1030x over previous
import functools
import math

import jax
import jax.numpy as jnp
from jax.experimental import pallas as pl
from jax.experimental.pallas import tpu as pltpu

_EPS = 1e-5
_VMEM = 64 * 1024 * 1024


def _lstm_cell(z, c_prev, hp):
    i = jax.nn.sigmoid(z[:, 0 * hp:1 * hp])
    f = jax.nn.sigmoid(z[:, 1 * hp:2 * hp])
    g = jnp.tanh(z[:, 2 * hp:3 * hp])
    o = jax.nn.sigmoid(z[:, 3 * hp:4 * hp])
    c = f * c_prev + i * g
    return o * jnp.tanh(c), c


def _feat_kernel(*refs, T, B, dims):
    (xt, w1t, b1t, whh1t, w2t, b2t, lngt, lnbt, whh2t,
     xv, w1v, b1v, whh1v, w2v, b2v, lngv, lnbv, whh2v,
     xa, w1a, b1a, whh1a, w2a, b2a, lnga, lnba, whh2a,
     ut_ref, uv_ref, ua_ref,
     gt, gv, ga, h1t, h1v, h1a,
     w2tb, w2vp, w2ap) = refs

    mods = []
    for (H, Hp), g_scr, h1, whh1, whh2, x, w1, b1, w2, b2, lng, \
            lnb, out in (
            (dims[0], gt, h1t, whh1t, whh2t, xt, w1t, b1t, w2t,
             b2t, lngt, lnbt, ut_ref),
            (dims[1], gv, h1v, whh1v, whh2v, xv, w1v, b1v, w2v,
             b2v, lngv, lnbv, uv_ref),
            (dims[2], ga, h1a, whh1a, whh2a, xa, w1a, b1a, w2a,
             b2a, lnga, lnba, ua_ref)):
        mods.append(dict(H=H, Hp=Hp, g=g_scr, h1=h1,
                         whh1=whh1, whh2=whh2, x=x, w1=w1, b1=b1, w2=w2,
                         b2=b2, lng=lng, lnb=lnb, out=out))

    w2scrs = (w2tb, w2vp, w2ap)
    for scr, m in zip(w2scrs, mods):
        H, Hp = m["H"], m["Hp"]
        N = scr.shape[1]
        if 2 * H >= Hp:
            scr[0:H, :] = m["w2"][H:2 * H, :]
            scr[H:Hp, :] = jnp.zeros((Hp - H, N), jnp.float32)
        else:
            scr[0:H, :] = m["w2"][0:H, :]
            scr[H:Hp, :] = jnp.zeros((Hp - H, N), jnp.float32)
            scr[Hp:Hp + H, :] = m["w2"][H:2 * H, :]
            scr[Hp + H:2 * Hp, :] = jnp.zeros((Hp - H, N), jnp.float32)

    for m in mods:
        m["g"][...] = (
            jnp.dot(m["x"][...], m["w1"][...],
                    preferred_element_type=jnp.float32) + m["b1"][...]
        )

    def run_layer(whh_key, store_h):
        st = [None] * len(mods)
        for t in range(T):
            zs = []
            for i, m in enumerate(mods):
                Hp, G4 = m["Hp"], 4 * m["Hp"]
                gf = m["g"][t * B:(t + 1) * B, 0:G4]
                gb = m["g"][(T - 1 - t) * B:(T - t) * B, G4:2 * G4]
                if t == 0:
                    zs.append((gf, gb))
                else:
                    zf = gf + jnp.dot(st[i][0], m[whh_key][0],
                                      preferred_element_type=jnp.float32)
                    zb = gb + jnp.dot(st[i][1], m[whh_key][1],
                                      preferred_element_type=jnp.float32)
                    zs.append((zf, zb))
            for i, (m, (zf, zb)) in enumerate(zip(mods, zs)):
                Hp = m["Hp"]
                if t == 0:
                    i_f = jax.nn.sigmoid(zf[:, 0:Hp])
                    g_f = jnp.tanh(zf[:, 2 * Hp:3 * Hp])
                    o_f = jax.nn.sigmoid(zf[:, 3 * Hp:4 * Hp])
                    cf = i_f * g_f
                    hf = o_f * jnp.tanh(cf)
                    i_b = jax.nn.sigmoid(zb[:, 0:Hp])
                    g_b = jnp.tanh(zb[:, 2 * Hp:3 * Hp])
                    o_b = jax.nn.sigmoid(zb[:, 3 * Hp:4 * Hp])
                    cb = i_b * g_b
                    hb = o_b * jnp.tanh(cb)
                else:
                    hf, cf = _lstm_cell(zf, st[i][2], Hp)
                    hb, cb = _lstm_cell(zb, st[i][3], Hp)
                st[i] = (hf, hb, cf, cb)
                if store_h:
                    m["h1"][t * B:(t + 1) * B, 0:Hp] = hf
                    m["h1"][(T - 1 - t) * B:(T - t) * B, Hp:2 * Hp] = hb
        return st

    st1 = run_layer("whh1", store_h=True)
    for m, (hf, hb, _, _) in zip(mods, st1):
        Hp = m["Hp"]
        m["out"][:, 0 * Hp:1 * Hp] = hf
        m["out"][:, 2 * Hp:3 * Hp] = hb

    for scr, m in zip(w2scrs, mods):
        H, Hp = m["H"], m["Hp"]
        x1 = m["h1"][...]
        inv_n = 1.0 / (2 * H)
        mu = jnp.sum(x1, axis=1, keepdims=True) * inv_n
        ex2 = jnp.sum(x1 * x1, axis=1, keepdims=True) * inv_n
        xn = (x1 - mu) * jax.lax.rsqrt(ex2 - mu * mu + _EPS) * \
            m["lng"][...] + m["lnb"][...]
        if 2 * H >= Hp:
            m["g"][...] = (
                jnp.dot(xn[:, 0:Hp], m["w2"][0:Hp, :],
                        preferred_element_type=jnp.float32)
                + jnp.dot(xn[:, Hp:2 * Hp], scr[...],
                          preferred_element_type=jnp.float32)
                + m["b2"][...]
            )
        else:
            m["g"][...] = (
                jnp.dot(xn, scr[...], preferred_element_type=jnp.float32)
                + m["b2"][...]
            )

    st2 = run_layer("whh2", store_h=False)
    for m, (hf, hb, _, _) in zip(mods, st2):
        Hp = m["Hp"]
        m["out"][:, 1 * Hp:2 * Hp] = hf
        m["out"][:, 3 * Hp:4 * Hp] = hb


def _head_kernel(*refs, nhead, dims):
    (ut, uv, ua,
     pwt, pbt, pgt, ptt, pwv, pbv, pgv, ptv, pwa, pba, pga, pta,
     qtw, qtb, qvw, qvb, qaw, qab, shw, shb, sdw, sdb,
     inw, inb, ouw, oub, l1g, l1b, f1w, f1b, f2w, f2b, l2g, l2b,
     fw, fb, o_ref, st_ref, sv_ref, sa_ref, ss_ref,
     pts, pvs, pas) = refs

    E = shw.shape[0]
    B = ut.shape[0]
    S = 6
    SB = S * B
    dh = E // nhead
    scale = 1.0 / math.sqrt(dh)

    for scr, w, (H, Hp) in ((pts, pwt, dims[0]), (pvs, pwv, dims[1]),
                            (pas, pwa, dims[2])):
        if H == Hp:
            scr[...] = w[...]
        else:
            for k in range(4):
                scr[k * Hp:k * Hp + H, :] = w[k * H:(k + 1) * H, :]
                scr[k * Hp + H:(k + 1) * Hp, :] = jnp.zeros(
                    (Hp - H, E), jnp.float32)
    pwt, pwv, pwa = pts, pvs, pas

    def ln(x, g, b):
        mu = jnp.mean(x, axis=-1, keepdims=True)
        xc = x - mu
        var = jnp.mean(xc * xc, axis=-1, keepdims=True)
        return xc * jax.lax.rsqrt(var + _EPS) * g[...] + b[...]

    def lin(x, w, b):
        return jnp.dot(x, w[...], preferred_element_type=jnp.float32) + b[...]

    t = ln(jnp.maximum(lin(ut[...], pwt, pbt), 0.0), pgt, ptt)
    v = ln(jnp.maximum(lin(uv[...], pwv, pbv), 0.0), pgv, ptv)
    a = ln(jnp.maximum(lin(ua[...], pwa, pba), 0.0), pga, pta)

    p_t = jax.nn.sigmoid(lin(t, qtw, qtb))
    p_v = jax.nn.sigmoid(lin(v, qvw, qvb))
    p_a = jax.nn.sigmoid(lin(a, qaw, qab))
    s_t = jax.nn.sigmoid(lin(t, shw, shb))
    s_v = jax.nn.sigmoid(lin(v, shw, shb))
    s_a = jax.nn.sigmoid(lin(a, shw, shb))

    st_ref[...] = lin(p_t, sdw, sdb)
    sv_ref[...] = lin(p_v, sdw, sdb)
    sa_ref[...] = lin(p_a, sdw, sdb)
    ss_ref[...] = lin((s_t + s_v + s_a) / 3.0, sdw, sdb)

    h = jnp.concatenate([p_t, p_v, p_a, s_t, s_v, s_a], axis=0)

    qkv = lin(h, inw, inb)
    q, k, vv = qkv[:, :E], qkv[:, E:2 * E], qkv[:, 2 * E:]
    ri = jax.lax.broadcasted_iota(jnp.int32, (SB, SB), 0)
    rj = jax.lax.broadcasted_iota(jnp.int32, (SB, SB), 1)
    same = (ri % B) == (rj % B)

    attn = jnp.zeros((SB, E), jnp.float32)
    for hd in range(nhead):
        cs = slice(hd * dh, (hd + 1) * dh)
        sc = jax.lax.dot_general(
            q[:, cs], k[:, cs], dimension_numbers=(((1,), (1,)), ((), ())),
            preferred_element_type=jnp.float32) * scale
        sc = jnp.where(same, sc, -1e30)
        m = jnp.max(sc, axis=-1, keepdims=True)
        p = jnp.exp(sc - m)
        p = p / jnp.sum(p, axis=-1, keepdims=True)
        hv = jnp.dot(p, vv[:, cs], preferred_element_type=jnp.float32)
        attn = attn + jnp.dot(hv, ouw[cs, :],
                              preferred_element_type=jnp.float32)

    x = ln(h + attn + oub[...], l1g, l1b)
    x = ln(x + lin(jnp.maximum(lin(x, f1w, f1b), 0.0), f2w, f2b), l2g, l2b)

    o = jnp.zeros((B, fw.shape[1]), jnp.float32)
    for s in range(S):
        o = o + jnp.dot(x[s * B:(s + 1) * B, :], fw[s * E:(s + 1) * E, :],
                        preferred_element_type=jnp.float32)
    o_ref[...] = o + fb[...]


def _pad_vec(g, H, Hp):
    if H == Hp:
        return g.reshape(1, -1)
    z = jnp.zeros((Hp - H,), g.dtype)
    return jnp.concatenate([g[:H], z, g[H:], z]).reshape(1, -1)


def kernel(trnn1_w_ih, trnn1_b, trnn1_w_hh, trnn2_w_ih, trnn2_b, trnn2_w_hh,
           vrnn1_w_ih, vrnn1_b, vrnn1_w_hh, vrnn2_w_ih, vrnn2_b, vrnn2_w_hh,
           arnn1_w_ih, arnn1_b, arnn1_w_hh, arnn2_w_ih, arnn2_b, arnn2_w_hh,
           tln_g, tln_b, vln_g, vln_b, aln_g, aln_b,
           proj_t_w, proj_t_b, proj_t_ln_g, proj_t_ln_b,
           proj_v_w, proj_v_b, proj_v_ln_g, proj_v_ln_b,
           proj_a_w, proj_a_b, proj_a_ln_g, proj_a_ln_b,
           priv_t_w, priv_t_b, priv_v_w, priv_v_b, priv_a_w, priv_a_b,
           shared_w, shared_b, spd_w, spd_b, fusion_w, fusion_b,
           tx_in_w, tx_in_b, tx_out_w, tx_out_b,
           tx_ff1_w, tx_ff1_b, tx_ff2_w, tx_ff2_b,
           tx_ln1_g, tx_ln1_b, tx_ln2_g, tx_ln2_b,
           visual, acoustic, sentences):
    B, T, Ht = sentences.shape
    Hv = visual.shape[2]
    Ha = acoustic.shape[2]
    Hpt = trnn1_w_hh.shape[1]
    Hpv = vrnn1_w_hh.shape[1]
    Hpa = arnn1_w_hh.shape[1]

    xt = jnp.transpose(sentences, (1, 0, 2)).reshape(T * B, Ht)
    xv = jnp.transpose(visual, (1, 0, 2)).reshape(T * B, Hv)
    xa = jnp.transpose(acoustic, (1, 0, 2)).reshape(T * B, Ha)

    r = lambda z: z.reshape(1, -1)
    feat_in = (
        xt, trnn1_w_ih, r(trnn1_b), trnn1_w_hh,
        trnn2_w_ih, r(trnn2_b),
        _pad_vec(tln_g, Ht, Hpt), _pad_vec(tln_b, Ht, Hpt), trnn2_w_hh,
        xv, vrnn1_w_ih, r(vrnn1_b), vrnn1_w_hh,
        vrnn2_w_ih, r(vrnn2_b),
        _pad_vec(vln_g, Hv, Hpv), _pad_vec(vln_b, Hv, Hpv), vrnn2_w_hh,
        xa, arnn1_w_ih, r(arnn1_b), arnn1_w_hh,
        arnn2_w_ih, r(arnn2_b),
        _pad_vec(aln_g, Ha, Hpa), _pad_vec(aln_b, Ha, Hpa), arnn2_w_hh,
    )

    ut, uv, ua = pl.pallas_call(
        functools.partial(_feat_kernel, T=T, B=B,
                          dims=((Ht, Hpt), (Hv, Hpv), (Ha, Hpa))),
        out_shape=[
            jax.ShapeDtypeStruct((B, 4 * Hpt), jnp.float32),
            jax.ShapeDtypeStruct((B, 4 * Hpv), jnp.float32),
            jax.ShapeDtypeStruct((B, 4 * Hpa), jnp.float32),
        ],
        scratch_shapes=[
            pltpu.VMEM((T * B, 8 * Hpt), jnp.float32),
            pltpu.VMEM((T * B, 8 * Hpv), jnp.float32),
            pltpu.VMEM((T * B, 8 * Hpa), jnp.float32),
            pltpu.VMEM((T * B, 2 * Hpt), jnp.float32),
            pltpu.VMEM((T * B, 2 * Hpv), jnp.float32),
            pltpu.VMEM((T * B, 2 * Hpa), jnp.float32),
            pltpu.VMEM((Hpt if 2 * Ht >= Hpt else 2 * Hpt, 8 * Hpt),
                       jnp.float32),
            pltpu.VMEM((Hpv if 2 * Hv >= Hpv else 2 * Hpv, 8 * Hpv),
                       jnp.float32),
            pltpu.VMEM((Hpa if 2 * Ha >= Hpa else 2 * Hpa, 8 * Hpa),
                       jnp.float32),
        ],
        compiler_params=pltpu.CompilerParams(vmem_limit_bytes=_VMEM),
    )(*feat_in)

    head_in = (
        ut, uv, ua,
        proj_t_w, r(proj_t_b),
        r(proj_t_ln_g), r(proj_t_ln_b),
        proj_v_w, r(proj_v_b),
        r(proj_v_ln_g), r(proj_v_ln_b),
        proj_a_w, r(proj_a_b),
        r(proj_a_ln_g), r(proj_a_ln_b),
        priv_t_w, r(priv_t_b), priv_v_w, r(priv_v_b), priv_a_w, r(priv_a_b),
        shared_w, r(shared_b), spd_w, r(spd_b),
        tx_in_w, r(tx_in_b), tx_out_w, r(tx_out_b),
        r(tx_ln1_g), r(tx_ln1_b),
        tx_ff1_w, r(tx_ff1_b), tx_ff2_w, r(tx_ff2_b),
        r(tx_ln2_g), r(tx_ln2_b),
        fusion_w, r(fusion_b),
    )
    E = shared_w.shape[0]
    o, spt, spv, spa, sps = pl.pallas_call(
        functools.partial(_head_kernel, nhead=2,
                          dims=((Ht, Hpt), (Hv, Hpv), (Ha, Hpa))),
        out_shape=(
            jax.ShapeDtypeStruct((B, 3 * E), jnp.float32),
            jax.ShapeDtypeStruct((B, 4), jnp.float32),
            jax.ShapeDtypeStruct((B, 4), jnp.float32),
            jax.ShapeDtypeStruct((B, 4), jnp.float32),
            jax.ShapeDtypeStruct((B, 4), jnp.float32),
        ),
        scratch_shapes=[
            pltpu.VMEM((4 * Hpt, E), jnp.float32),
            pltpu.VMEM((4 * Hpv, E), jnp.float32),
            pltpu.VMEM((4 * Hpa, E), jnp.float32),
        ],
        compiler_params=pltpu.CompilerParams(vmem_limit_bytes=_VMEM),
    )(*head_in)
    aux = {"sp_p_t": spt, "sp_p_v": spv, "sp_p_a": spa, "sp_s": sps}
    return o, aux


# speedup vs baseline: 1.1196x; 1.0150x over previous
import functools
import math

import jax
import jax.numpy as jnp
from jax.experimental import pallas as pl
from jax.experimental.pallas import tpu as pltpu

_EPS = 1e-5
_VMEM = 64 * 1024 * 1024


def _lstm_cell(z, c_prev, hp):
    i = jax.nn.sigmoid(z[:, 0 * hp:1 * hp])
    f = jax.nn.sigmoid(z[:, 1 * hp:2 * hp])
    g = jnp.tanh(z[:, 2 * hp:3 * hp])
    o = jax.nn.sigmoid(z[:, 3 * hp:4 * hp])
    c = f * c_prev + i * g
    return o * jnp.tanh(c), c


def _feat_kernel(*refs, T, B, dims):
    (xt, w1t, b1t, whh1t, w2t, b2t, lngt, lnbt, whh2t,
     xv, w1v, b1v, whh1v, w2v, b2v, lngv, lnbv, whh2v,
     xa, w1a, b1a, whh1a, w2a, b2a, lnga, lnba, whh2a,
     ut_ref, uv_ref, ua_ref,
     gt, gv, ga, h1t, h1v, h1a,
     w2tb, w2vp, w2ap, p_scr, xtm, xvm, xam) = refs

    mods = []
    for (H, Hp), g_scr, h1, whh1, whh2, x, w1, b1, w2, b2, lng, \
            lnb, out in (
            (dims[0], gt, h1t, whh1t, whh2t, xt, w1t, b1t, w2t,
             b2t, lngt, lnbt, ut_ref),
            (dims[1], gv, h1v, whh1v, whh2v, xv, w1v, b1v, w2v,
             b2v, lngv, lnbv, uv_ref),
            (dims[2], ga, h1a, whh1a, whh2a, xa, w1a, b1a, w2a,
             b2a, lnga, lnba, ua_ref)):
        mods.append(dict(H=H, Hp=Hp, g=g_scr, h1=h1,
                         whh1=whh1, whh2=whh2, x=x, w1=w1, b1=b1, w2=w2,
                         b2=b2, lng=lng, lnb=lnb, out=out))

    w2scrs = (w2tb, w2vp, w2ap)
    for scr, m in zip(w2scrs, mods):
        H, Hp = m["H"], m["Hp"]
        N = scr.shape[1]
        if 2 * H >= Hp:
            scr[0:H, :] = m["w2"][H:2 * H, :]
            scr[H:Hp, :] = jnp.zeros((Hp - H, N), jnp.float32)
        else:
            scr[0:H, :] = m["w2"][0:H, :]
            scr[H:Hp, :] = jnp.zeros((Hp - H, N), jnp.float32)
            scr[Hp:Hp + H, :] = m["w2"][H:2 * H, :]
            scr[Hp + H:2 * Hp, :] = jnp.zeros((Hp - H, N), jnp.float32)

    TB = T * B
    ri = jax.lax.broadcasted_iota(jnp.int32, (TB, TB), 0)
    ci = jax.lax.broadcasted_iota(jnp.int32, (TB, TB), 1)
    p_scr[...] = jnp.where(ci == (ri % B) * T + ri // B, 1.0, 0.0)

    for m, xm in zip(mods, (xtm, xvm, xam)):
        xm[...] = jnp.dot(p_scr[...], m["x"][...],
                          preferred_element_type=jnp.float32)
        m["g"][...] = (
            jnp.dot(xm[...], m["w1"][...],
                    preferred_element_type=jnp.float32) + m["b1"][...]
        )

    def run_layer(whh_key, store_h):
        st = [None] * len(mods)
        for t in range(T):
            zs = []
            for i, m in enumerate(mods):
                Hp, G4 = m["Hp"], 4 * m["Hp"]
                gf = m["g"][t * B:(t + 1) * B, 0:G4]
                gb = m["g"][(T - 1 - t) * B:(T - t) * B, G4:2 * G4]
                if t == 0:
                    zs.append((gf, gb))
                else:
                    zf = gf + jnp.dot(st[i][0], m[whh_key][0],
                                      preferred_element_type=jnp.float32)
                    zb = gb + jnp.dot(st[i][1], m[whh_key][1],
                                      preferred_element_type=jnp.float32)
                    zs.append((zf, zb))
            for i, (m, (zf, zb)) in enumerate(zip(mods, zs)):
                Hp = m["Hp"]
                if t == 0:
                    i_f = jax.nn.sigmoid(zf[:, 0:Hp])
                    g_f = jnp.tanh(zf[:, 2 * Hp:3 * Hp])
                    o_f = jax.nn.sigmoid(zf[:, 3 * Hp:4 * Hp])
                    cf = i_f * g_f
                    hf = o_f * jnp.tanh(cf)
                    i_b = jax.nn.sigmoid(zb[:, 0:Hp])
                    g_b = jnp.tanh(zb[:, 2 * Hp:3 * Hp])
                    o_b = jax.nn.sigmoid(zb[:, 3 * Hp:4 * Hp])
                    cb = i_b * g_b
                    hb = o_b * jnp.tanh(cb)
                else:
                    hf, cf = _lstm_cell(zf, st[i][2], Hp)
                    hb, cb = _lstm_cell(zb, st[i][3], Hp)
                st[i] = (hf, hb, cf, cb)
                if store_h:
                    m["h1"][t * B:(t + 1) * B, 0:Hp] = hf
                    m["h1"][(T - 1 - t) * B:(T - t) * B, Hp:2 * Hp] = hb
        return st

    st1 = run_layer("whh1", store_h=True)
    for m, (hf, hb, _, _) in zip(mods, st1):
        Hp = m["Hp"]
        m["out"][:, 0 * Hp:1 * Hp] = hf
        m["out"][:, 2 * Hp:3 * Hp] = hb

    for scr, m in zip(w2scrs, mods):
        H, Hp = m["H"], m["Hp"]
        x1 = m["h1"][...]
        inv_n = 1.0 / (2 * H)
        mu = jnp.sum(x1, axis=1, keepdims=True) * inv_n
        ex2 = jnp.sum(x1 * x1, axis=1, keepdims=True) * inv_n
        xn = (x1 - mu) * jax.lax.rsqrt(ex2 - mu * mu + _EPS) * \
            m["lng"][...] + m["lnb"][...]
        if 2 * H >= Hp:
            m["g"][...] = (
                jnp.dot(xn[:, 0:Hp], m["w2"][0:Hp, :],
                        preferred_element_type=jnp.float32)
                + jnp.dot(xn[:, Hp:2 * Hp], scr[...],
                          preferred_element_type=jnp.float32)
                + m["b2"][...]
            )
        else:
            m["g"][...] = (
                jnp.dot(xn, scr[...], preferred_element_type=jnp.float32)
                + m["b2"][...]
            )

    st2 = run_layer("whh2", store_h=False)
    for m, (hf, hb, _, _) in zip(mods, st2):
        Hp = m["Hp"]
        m["out"][:, 1 * Hp:2 * Hp] = hf
        m["out"][:, 3 * Hp:4 * Hp] = hb


def _head_kernel(*refs, nhead, dims):
    (ut, uv, ua,
     pwt, pbt, pgt, ptt, pwv, pbv, pgv, ptv, pwa, pba, pga, pta,
     qtw, qtb, qvw, qvb, qaw, qab, shw, shb, sdw, sdb,
     inw, inb, ouw, oub, l1g, l1b, f1w, f1b, f2w, f2b, l2g, l2b,
     fw, fb, o_ref, st_ref, sv_ref, sa_ref, ss_ref,
     pts, pvs, pas) = refs

    E = shw.shape[0]
    B = ut.shape[0]
    S = 6
    SB = S * B
    dh = E // nhead
    scale = 1.0 / math.sqrt(dh)

    for scr, w, (H, Hp) in ((pts, pwt, dims[0]), (pvs, pwv, dims[1]),
                            (pas, pwa, dims[2])):
        if H == Hp:
            scr[...] = w[...]
        else:
            for k in range(4):
                scr[k * Hp:k * Hp + H, :] = w[k * H:(k + 1) * H, :]
                scr[k * Hp + H:(k + 1) * Hp, :] = jnp.zeros(
                    (Hp - H, E), jnp.float32)
    pwt, pwv, pwa = pts, pvs, pas

    def ln(x, g, b):
        mu = jnp.mean(x, axis=-1, keepdims=True)
        xc = x - mu
        var = jnp.mean(xc * xc, axis=-1, keepdims=True)
        return xc * jax.lax.rsqrt(var + _EPS) * g[...] + b[...]

    def lin(x, w, b):
        return jnp.dot(x, w[...], preferred_element_type=jnp.float32) + b[...]

    t = ln(jnp.maximum(lin(ut[...], pwt, pbt), 0.0), pgt, ptt)
    v = ln(jnp.maximum(lin(uv[...], pwv, pbv), 0.0), pgv, ptv)
    a = ln(jnp.maximum(lin(ua[...], pwa, pba), 0.0), pga, pta)

    p_t = jax.nn.sigmoid(lin(t, qtw, qtb))
    p_v = jax.nn.sigmoid(lin(v, qvw, qvb))
    p_a = jax.nn.sigmoid(lin(a, qaw, qab))
    s_t = jax.nn.sigmoid(lin(t, shw, shb))
    s_v = jax.nn.sigmoid(lin(v, shw, shb))
    s_a = jax.nn.sigmoid(lin(a, shw, shb))

    st_ref[...] = lin(p_t, sdw, sdb)
    sv_ref[...] = lin(p_v, sdw, sdb)
    sa_ref[...] = lin(p_a, sdw, sdb)
    ss_ref[...] = lin((s_t + s_v + s_a) / 3.0, sdw, sdb)

    h = jnp.concatenate([p_t, p_v, p_a, s_t, s_v, s_a], axis=0)

    qkv = lin(h, inw, inb)
    q, k, vv = qkv[:, :E], qkv[:, E:2 * E], qkv[:, 2 * E:]
    ri = jax.lax.broadcasted_iota(jnp.int32, (SB, SB), 0)
    rj = jax.lax.broadcasted_iota(jnp.int32, (SB, SB), 1)
    same = (ri % B) == (rj % B)

    attn = jnp.zeros((SB, E), jnp.float32)
    for hd in range(nhead):
        cs = slice(hd * dh, (hd + 1) * dh)
        sc = jax.lax.dot_general(
            q[:, cs], k[:, cs], dimension_numbers=(((1,), (1,)), ((), ())),
            preferred_element_type=jnp.float32) * scale
        sc = jnp.where(same, sc, -1e30)
        m = jnp.max(sc, axis=-1, keepdims=True)
        p = jnp.exp(sc - m)
        p = p / jnp.sum(p, axis=-1, keepdims=True)
        hv = jnp.dot(p, vv[:, cs], preferred_element_type=jnp.float32)
        attn = attn + jnp.dot(hv, ouw[cs, :],
                              preferred_element_type=jnp.float32)

    x = ln(h + attn + oub[...], l1g, l1b)
    x = ln(x + lin(jnp.maximum(lin(x, f1w, f1b), 0.0), f2w, f2b), l2g, l2b)

    o = jnp.zeros((B, fw.shape[1]), jnp.float32)
    for s in range(S):
        o = o + jnp.dot(x[s * B:(s + 1) * B, :], fw[s * E:(s + 1) * E, :],
                        preferred_element_type=jnp.float32)
    o_ref[...] = o + fb[...]


def _pad_vec(g, H, Hp):
    if H == Hp:
        return g.reshape(1, -1)
    z = jnp.zeros((Hp - H,), g.dtype)
    return jnp.concatenate([g[:H], z, g[H:], z]).reshape(1, -1)


def kernel(trnn1_w_ih, trnn1_b, trnn1_w_hh, trnn2_w_ih, trnn2_b, trnn2_w_hh,
           vrnn1_w_ih, vrnn1_b, vrnn1_w_hh, vrnn2_w_ih, vrnn2_b, vrnn2_w_hh,
           arnn1_w_ih, arnn1_b, arnn1_w_hh, arnn2_w_ih, arnn2_b, arnn2_w_hh,
           tln_g, tln_b, vln_g, vln_b, aln_g, aln_b,
           proj_t_w, proj_t_b, proj_t_ln_g, proj_t_ln_b,
           proj_v_w, proj_v_b, proj_v_ln_g, proj_v_ln_b,
           proj_a_w, proj_a_b, proj_a_ln_g, proj_a_ln_b,
           priv_t_w, priv_t_b, priv_v_w, priv_v_b, priv_a_w, priv_a_b,
           shared_w, shared_b, spd_w, spd_b, fusion_w, fusion_b,
           tx_in_w, tx_in_b, tx_out_w, tx_out_b,
           tx_ff1_w, tx_ff1_b, tx_ff2_w, tx_ff2_b,
           tx_ln1_g, tx_ln1_b, tx_ln2_g, tx_ln2_b,
           visual, acoustic, sentences):
    B, T, Ht = sentences.shape
    Hv = visual.shape[2]
    Ha = acoustic.shape[2]
    Hpt = trnn1_w_hh.shape[1]
    Hpv = vrnn1_w_hh.shape[1]
    Hpa = arnn1_w_hh.shape[1]

    xt = sentences.reshape(B * T, Ht)
    xv = visual.reshape(B * T, Hv)
    xa = acoustic.reshape(B * T, Ha)

    r = lambda z: z.reshape(1, -1)
    feat_in = (
        xt, trnn1_w_ih, r(trnn1_b), trnn1_w_hh,
        trnn2_w_ih, r(trnn2_b),
        _pad_vec(tln_g, Ht, Hpt), _pad_vec(tln_b, Ht, Hpt), trnn2_w_hh,
        xv, vrnn1_w_ih, r(vrnn1_b), vrnn1_w_hh,
        vrnn2_w_ih, r(vrnn2_b),
        _pad_vec(vln_g, Hv, Hpv), _pad_vec(vln_b, Hv, Hpv), vrnn2_w_hh,
        xa, arnn1_w_ih, r(arnn1_b), arnn1_w_hh,
        arnn2_w_ih, r(arnn2_b),
        _pad_vec(aln_g, Ha, Hpa), _pad_vec(aln_b, Ha, Hpa), arnn2_w_hh,
    )

    ut, uv, ua = pl.pallas_call(
        functools.partial(_feat_kernel, T=T, B=B,
                          dims=((Ht, Hpt), (Hv, Hpv), (Ha, Hpa))),
        out_shape=[
            jax.ShapeDtypeStruct((B, 4 * Hpt), jnp.float32),
            jax.ShapeDtypeStruct((B, 4 * Hpv), jnp.float32),
            jax.ShapeDtypeStruct((B, 4 * Hpa), jnp.float32),
        ],
        scratch_shapes=[
            pltpu.VMEM((T * B, 8 * Hpt), jnp.float32),
            pltpu.VMEM((T * B, 8 * Hpv), jnp.float32),
            pltpu.VMEM((T * B, 8 * Hpa), jnp.float32),
            pltpu.VMEM((T * B, 2 * Hpt), jnp.float32),
            pltpu.VMEM((T * B, 2 * Hpv), jnp.float32),
            pltpu.VMEM((T * B, 2 * Hpa), jnp.float32),
            pltpu.VMEM((Hpt if 2 * Ht >= Hpt else 2 * Hpt, 8 * Hpt),
                       jnp.float32),
            pltpu.VMEM((Hpv if 2 * Hv >= Hpv else 2 * Hpv, 8 * Hpv),
                       jnp.float32),
            pltpu.VMEM((Hpa if 2 * Ha >= Hpa else 2 * Hpa, 8 * Hpa),
                       jnp.float32),
            pltpu.VMEM((T * B, T * B), jnp.float32),
            pltpu.VMEM((T * B, Ht), jnp.float32),
            pltpu.VMEM((T * B, Hv), jnp.float32),
            pltpu.VMEM((T * B, Ha), jnp.float32),
        ],
        compiler_params=pltpu.CompilerParams(vmem_limit_bytes=_VMEM),
    )(*feat_in)

    head_in = (
        ut, uv, ua,
        proj_t_w, r(proj_t_b),
        r(proj_t_ln_g), r(proj_t_ln_b),
        proj_v_w, r(proj_v_b),
        r(proj_v_ln_g), r(proj_v_ln_b),
        proj_a_w, r(proj_a_b),
        r(proj_a_ln_g), r(proj_a_ln_b),
        priv_t_w, r(priv_t_b), priv_v_w, r(priv_v_b), priv_a_w, r(priv_a_b),
        shared_w, r(shared_b), spd_w, r(spd_b),
        tx_in_w, r(tx_in_b), tx_out_w, r(tx_out_b),
        r(tx_ln1_g), r(tx_ln1_b),
        tx_ff1_w, r(tx_ff1_b), tx_ff2_w, r(tx_ff2_b),
        r(tx_ln2_g), r(tx_ln2_b),
        fusion_w, r(fusion_b),
    )
    E = shared_w.shape[0]
    o, spt, spv, spa, sps = pl.pallas_call(
        functools.partial(_head_kernel, nhead=2,
                          dims=((Ht, Hpt), (Hv, Hpv), (Ha, Hpa))),
        out_shape=(
            jax.ShapeDtypeStruct((B, 3 * E), jnp.float32),
            jax.ShapeDtypeStruct((B, 4), jnp.float32),
            jax.ShapeDtypeStruct((B, 4), jnp.float32),
            jax.ShapeDtypeStruct((B, 4), jnp.float32),
            jax.ShapeDtypeStruct((B, 4), jnp.float32),
        ),
        scratch_shapes=[
            pltpu.VMEM((4 * Hpt, E), jnp.float32),
            pltpu.VMEM((4 * Hpv, E), jnp.float32),
            pltpu.VMEM((4 * Hpa, E), jnp.float32),
        ],
        compiler_params=pltpu.CompilerParams(vmem_limit_bytes=_VMEM),
    )(*head_in)
    aux = {"sp_p_t": spt, "sp_p_v": spv, "sp_p_a": spa, "sp_s": sps}
    return o, aux


# speedup vs baseline: 1.1566x; 1.0330x over previous
import functools
import math

import jax
import jax.numpy as jnp
from jax.experimental import pallas as pl
from jax.experimental.pallas import tpu as pltpu

_EPS = 1e-5
_VMEM = 64 * 1024 * 1024


def _lstm_cell(z, c_prev, hp):
    i = jax.nn.sigmoid(z[:, 0 * hp:1 * hp])
    f = jax.nn.sigmoid(z[:, 1 * hp:2 * hp])
    g = jnp.tanh(z[:, 2 * hp:3 * hp])
    o = jax.nn.sigmoid(z[:, 3 * hp:4 * hp])
    c = f * c_prev + i * g
    return o * jnp.tanh(c), c


def _feat_kernel(*refs, T, B, dims):
    (xt, w1t, b1t, whh1t, w2t, b2t, lngt, lnbt, whh2t,
     xv, w1v, b1v, whh1v, w2v, b2v, lngv, lnbv, whh2v,
     xa, w1a, b1a, whh1a, w2a, b2a, lnga, lnba, whh2a,
     ut_ref, uv_ref, ua_ref,
     gt, gv, ga, h1t, h1v, h1a,
     w2tb, w2vp, w2ap, w1tb, w1vb, w1ab, p_scr, xtm, xvm, xam) = refs

    mods = []
    for (H, Hp), g_scr, h1, whh1, whh2, x, w1, b1, w2, b2, lng, \
            lnb, out in (
            (dims[0], gt, h1t, whh1t, whh2t, xt, w1t, b1t, w2t,
             b2t, lngt, lnbt, ut_ref),
            (dims[1], gv, h1v, whh1v, whh2v, xv, w1v, b1v, w2v,
             b2v, lngv, lnbv, uv_ref),
            (dims[2], ga, h1a, whh1a, whh2a, xa, w1a, b1a, w2a,
             b2a, lnga, lnba, ua_ref)):
        mods.append(dict(H=H, Hp=Hp, g=g_scr, h1=h1,
                         whh1=whh1, whh2=whh2, x=x, w1=w1, b1=b1, w2=w2,
                         b2=b2, lng=lng, lnb=lnb, out=out))

    bf = jnp.bfloat16
    w2scrs = (w2tb, w2vp, w2ap)
    for scr, m in zip(w2scrs, mods):
        H, Hp = m["H"], m["Hp"]
        N = scr.shape[1]
        scr[0:H, :] = m["w2"][0:H, :].astype(bf)
        scr[H:Hp, :] = jnp.zeros((Hp - H, N), bf)
        scr[Hp:Hp + H, :] = m["w2"][H:2 * H, :].astype(bf)
        scr[Hp + H:2 * Hp, :] = jnp.zeros((Hp - H, N), bf)
    w1scrs = (w1tb, w1vb, w1ab)
    for scr, m in zip(w1scrs, mods):
        scr[...] = m["w1"][...].astype(bf)

    TB = T * B
    ri = jax.lax.broadcasted_iota(jnp.int32, (TB, TB), 0)
    ci = jax.lax.broadcasted_iota(jnp.int32, (TB, TB), 1)
    p_scr[...] = jnp.where(ci == (ri % B) * T + ri // B, 1.0, 0.0)

    for m, xm, w1b in zip(mods, (xtm, xvm, xam), w1scrs):
        xm[...] = jnp.dot(p_scr[...], m["x"][...],
                          preferred_element_type=jnp.float32).astype(bf)
        m["g"][...] = (
            jnp.dot(xm[...], w1b[...],
                    preferred_element_type=jnp.float32) + m["b1"][...]
        )

    def run_layer(whh_key, store_h):
        st = [None] * len(mods)
        for t in range(T):
            zs = []
            for i, m in enumerate(mods):
                Hp, G4 = m["Hp"], 4 * m["Hp"]
                gf = m["g"][t * B:(t + 1) * B, 0:G4]
                gb = m["g"][(T - 1 - t) * B:(T - t) * B, G4:2 * G4]
                if t == 0:
                    zs.append((gf, gb))
                else:
                    zf = gf + jnp.dot(st[i][0], m[whh_key][0],
                                      preferred_element_type=jnp.float32)
                    zb = gb + jnp.dot(st[i][1], m[whh_key][1],
                                      preferred_element_type=jnp.float32)
                    zs.append((zf, zb))
            for i, (m, (zf, zb)) in enumerate(zip(mods, zs)):
                Hp = m["Hp"]
                if t == 0:
                    i_f = jax.nn.sigmoid(zf[:, 0:Hp])
                    g_f = jnp.tanh(zf[:, 2 * Hp:3 * Hp])
                    o_f = jax.nn.sigmoid(zf[:, 3 * Hp:4 * Hp])
                    cf = i_f * g_f
                    hf = o_f * jnp.tanh(cf)
                    i_b = jax.nn.sigmoid(zb[:, 0:Hp])
                    g_b = jnp.tanh(zb[:, 2 * Hp:3 * Hp])
                    o_b = jax.nn.sigmoid(zb[:, 3 * Hp:4 * Hp])
                    cb = i_b * g_b
                    hb = o_b * jnp.tanh(cb)
                else:
                    hf, cf = _lstm_cell(zf, st[i][2], Hp)
                    hb, cb = _lstm_cell(zb, st[i][3], Hp)
                st[i] = (hf, hb, cf, cb)
                if store_h:
                    m["h1"][t * B:(t + 1) * B, 0:Hp] = hf
                    m["h1"][(T - 1 - t) * B:(T - t) * B, Hp:2 * Hp] = hb
        return st

    st1 = run_layer("whh1", store_h=True)
    for m, (hf, hb, _, _) in zip(mods, st1):
        Hp = m["Hp"]
        m["out"][:, 0 * Hp:1 * Hp] = hf
        m["out"][:, 2 * Hp:3 * Hp] = hb

    for scr, m in zip(w2scrs, mods):
        H, Hp = m["H"], m["Hp"]
        x1 = m["h1"][...]
        inv_n = 1.0 / (2 * H)
        mu = jnp.sum(x1, axis=1, keepdims=True) * inv_n
        ex2 = jnp.sum(x1 * x1, axis=1, keepdims=True) * inv_n
        xn = (x1 - mu) * jax.lax.rsqrt(ex2 - mu * mu + _EPS) * \
            m["lng"][...] + m["lnb"][...]
        m["g"][...] = (
            jnp.dot(xn.astype(bf), scr[...],
                    preferred_element_type=jnp.float32)
            + m["b2"][...]
        )

    st2 = run_layer("whh2", store_h=False)
    for m, (hf, hb, _, _) in zip(mods, st2):
        Hp = m["Hp"]
        m["out"][:, 1 * Hp:2 * Hp] = hf
        m["out"][:, 3 * Hp:4 * Hp] = hb


def _head_kernel(*refs, nhead, dims):
    (ut, uv, ua,
     pwt, pbt, pgt, ptt, pwv, pbv, pgv, ptv, pwa, pba, pga, pta,
     qtw, qtb, qvw, qvb, qaw, qab, shw, shb, sdw, sdb,
     inw, inb, ouw, oub, l1g, l1b, f1w, f1b, f2w, f2b, l2g, l2b,
     fw, fb, o_ref, st_ref, sv_ref, sa_ref, ss_ref,
     pts, pvs, pas) = refs

    E = shw.shape[0]
    B = ut.shape[0]
    S = 6
    SB = S * B
    dh = E // nhead
    scale = 1.0 / math.sqrt(dh)

    for scr, w, (H, Hp) in ((pts, pwt, dims[0]), (pvs, pwv, dims[1]),
                            (pas, pwa, dims[2])):
        if H == Hp:
            scr[...] = w[...]
        else:
            for k in range(4):
                scr[k * Hp:k * Hp + H, :] = w[k * H:(k + 1) * H, :]
                scr[k * Hp + H:(k + 1) * Hp, :] = jnp.zeros(
                    (Hp - H, E), jnp.float32)
    pwt, pwv, pwa = pts, pvs, pas

    def ln(x, g, b):
        mu = jnp.mean(x, axis=-1, keepdims=True)
        xc = x - mu
        var = jnp.mean(xc * xc, axis=-1, keepdims=True)
        return xc * jax.lax.rsqrt(var + _EPS) * g[...] + b[...]

    def lin(x, w, b):
        return jnp.dot(x, w[...], preferred_element_type=jnp.float32) + b[...]

    t = ln(jnp.maximum(lin(ut[...], pwt, pbt), 0.0), pgt, ptt)
    v = ln(jnp.maximum(lin(uv[...], pwv, pbv), 0.0), pgv, ptv)
    a = ln(jnp.maximum(lin(ua[...], pwa, pba), 0.0), pga, pta)

    p_t = jax.nn.sigmoid(lin(t, qtw, qtb))
    p_v = jax.nn.sigmoid(lin(v, qvw, qvb))
    p_a = jax.nn.sigmoid(lin(a, qaw, qab))
    s_t = jax.nn.sigmoid(lin(t, shw, shb))
    s_v = jax.nn.sigmoid(lin(v, shw, shb))
    s_a = jax.nn.sigmoid(lin(a, shw, shb))

    st_ref[...] = lin(p_t, sdw, sdb)
    sv_ref[...] = lin(p_v, sdw, sdb)
    sa_ref[...] = lin(p_a, sdw, sdb)
    ss_ref[...] = lin((s_t + s_v + s_a) / 3.0, sdw, sdb)

    h = jnp.concatenate([p_t, p_v, p_a, s_t, s_v, s_a], axis=0)

    qkv = lin(h, inw, inb)
    q, k, vv = qkv[:, :E], qkv[:, E:2 * E], qkv[:, 2 * E:]
    ri = jax.lax.broadcasted_iota(jnp.int32, (SB, SB), 0)
    rj = jax.lax.broadcasted_iota(jnp.int32, (SB, SB), 1)
    same = (ri % B) == (rj % B)

    attn = jnp.zeros((SB, E), jnp.float32)
    for hd in range(nhead):
        cs = slice(hd * dh, (hd + 1) * dh)
        sc = jax.lax.dot_general(
            q[:, cs], k[:, cs], dimension_numbers=(((1,), (1,)), ((), ())),
            preferred_element_type=jnp.float32) * scale
        sc = jnp.where(same, sc, -1e30)
        m = jnp.max(sc, axis=-1, keepdims=True)
        p = jnp.exp(sc - m)
        p = p / jnp.sum(p, axis=-1, keepdims=True)
        hv = jnp.dot(p, vv[:, cs], preferred_element_type=jnp.float32)
        attn = attn + jnp.dot(hv, ouw[cs, :],
                              preferred_element_type=jnp.float32)

    x = ln(h + attn + oub[...], l1g, l1b)
    x = ln(x + lin(jnp.maximum(lin(x, f1w, f1b), 0.0), f2w, f2b), l2g, l2b)

    o = jnp.zeros((B, fw.shape[1]), jnp.float32)
    for s in range(S):
        o = o + jnp.dot(x[s * B:(s + 1) * B, :], fw[s * E:(s + 1) * E, :],
                        preferred_element_type=jnp.float32)
    o_ref[...] = o + fb[...]


def _pad_vec(g, H, Hp):
    if H == Hp:
        return g.reshape(1, -1)
    z = jnp.zeros((Hp - H,), g.dtype)
    return jnp.concatenate([g[:H], z, g[H:], z]).reshape(1, -1)


def kernel(trnn1_w_ih, trnn1_b, trnn1_w_hh, trnn2_w_ih, trnn2_b, trnn2_w_hh,
           vrnn1_w_ih, vrnn1_b, vrnn1_w_hh, vrnn2_w_ih, vrnn2_b, vrnn2_w_hh,
           arnn1_w_ih, arnn1_b, arnn1_w_hh, arnn2_w_ih, arnn2_b, arnn2_w_hh,
           tln_g, tln_b, vln_g, vln_b, aln_g, aln_b,
           proj_t_w, proj_t_b, proj_t_ln_g, proj_t_ln_b,
           proj_v_w, proj_v_b, proj_v_ln_g, proj_v_ln_b,
           proj_a_w, proj_a_b, proj_a_ln_g, proj_a_ln_b,
           priv_t_w, priv_t_b, priv_v_w, priv_v_b, priv_a_w, priv_a_b,
           shared_w, shared_b, spd_w, spd_b, fusion_w, fusion_b,
           tx_in_w, tx_in_b, tx_out_w, tx_out_b,
           tx_ff1_w, tx_ff1_b, tx_ff2_w, tx_ff2_b,
           tx_ln1_g, tx_ln1_b, tx_ln2_g, tx_ln2_b,
           visual, acoustic, sentences):
    B, T, Ht = sentences.shape
    Hv = visual.shape[2]
    Ha = acoustic.shape[2]
    Hpt = trnn1_w_hh.shape[1]
    Hpv = vrnn1_w_hh.shape[1]
    Hpa = arnn1_w_hh.shape[1]

    xt = sentences.reshape(B * T, Ht)
    xv = visual.reshape(B * T, Hv)
    xa = acoustic.reshape(B * T, Ha)

    r = lambda z: z.reshape(1, -1)
    feat_in = (
        xt, trnn1_w_ih, r(trnn1_b), trnn1_w_hh,
        trnn2_w_ih, r(trnn2_b),
        _pad_vec(tln_g, Ht, Hpt), _pad_vec(tln_b, Ht, Hpt), trnn2_w_hh,
        xv, vrnn1_w_ih, r(vrnn1_b), vrnn1_w_hh,
        vrnn2_w_ih, r(vrnn2_b),
        _pad_vec(vln_g, Hv, Hpv), _pad_vec(vln_b, Hv, Hpv), vrnn2_w_hh,
        xa, arnn1_w_ih, r(arnn1_b), arnn1_w_hh,
        arnn2_w_ih, r(arnn2_b),
        _pad_vec(aln_g, Ha, Hpa), _pad_vec(aln_b, Ha, Hpa), arnn2_w_hh,
    )

    ut, uv, ua = pl.pallas_call(
        functools.partial(_feat_kernel, T=T, B=B,
                          dims=((Ht, Hpt), (Hv, Hpv), (Ha, Hpa))),
        out_shape=[
            jax.ShapeDtypeStruct((B, 4 * Hpt), jnp.float32),
            jax.ShapeDtypeStruct((B, 4 * Hpv), jnp.float32),
            jax.ShapeDtypeStruct((B, 4 * Hpa), jnp.float32),
        ],
        scratch_shapes=[
            pltpu.VMEM((T * B, 8 * Hpt), jnp.float32),
            pltpu.VMEM((T * B, 8 * Hpv), jnp.float32),
            pltpu.VMEM((T * B, 8 * Hpa), jnp.float32),
            pltpu.VMEM((T * B, 2 * Hpt), jnp.float32),
            pltpu.VMEM((T * B, 2 * Hpv), jnp.float32),
            pltpu.VMEM((T * B, 2 * Hpa), jnp.float32),
            pltpu.VMEM((2 * Hpt, 8 * Hpt), jnp.bfloat16),
            pltpu.VMEM((2 * Hpv, 8 * Hpv), jnp.bfloat16),
            pltpu.VMEM((2 * Hpa, 8 * Hpa), jnp.bfloat16),
            pltpu.VMEM((Ht, 8 * Hpt), jnp.bfloat16),
            pltpu.VMEM((Hv, 8 * Hpv), jnp.bfloat16),
            pltpu.VMEM((Ha, 8 * Hpa), jnp.bfloat16),
            pltpu.VMEM((T * B, T * B), jnp.float32),
            pltpu.VMEM((T * B, Ht), jnp.bfloat16),
            pltpu.VMEM((T * B, Hv), jnp.bfloat16),
            pltpu.VMEM((T * B, Ha), jnp.bfloat16),
        ],
        compiler_params=pltpu.CompilerParams(vmem_limit_bytes=_VMEM),
    )(*feat_in)

    head_in = (
        ut, uv, ua,
        proj_t_w, r(proj_t_b),
        r(proj_t_ln_g), r(proj_t_ln_b),
        proj_v_w, r(proj_v_b),
        r(proj_v_ln_g), r(proj_v_ln_b),
        proj_a_w, r(proj_a_b),
        r(proj_a_ln_g), r(proj_a_ln_b),
        priv_t_w, r(priv_t_b), priv_v_w, r(priv_v_b), priv_a_w, r(priv_a_b),
        shared_w, r(shared_b), spd_w, r(spd_b),
        tx_in_w, r(tx_in_b), tx_out_w, r(tx_out_b),
        r(tx_ln1_g), r(tx_ln1_b),
        tx_ff1_w, r(tx_ff1_b), tx_ff2_w, r(tx_ff2_b),
        r(tx_ln2_g), r(tx_ln2_b),
        fusion_w, r(fusion_b),
    )
    E = shared_w.shape[0]
    o, spt, spv, spa, sps = pl.pallas_call(
        functools.partial(_head_kernel, nhead=2,
                          dims=((Ht, Hpt), (Hv, Hpv), (Ha, Hpa))),
        out_shape=(
            jax.ShapeDtypeStruct((B, 3 * E), jnp.float32),
            jax.ShapeDtypeStruct((B, 4), jnp.float32),
            jax.ShapeDtypeStruct((B, 4), jnp.float32),
            jax.ShapeDtypeStruct((B, 4), jnp.float32),
            jax.ShapeDtypeStruct((B, 4), jnp.float32),
        ),
        scratch_shapes=[
            pltpu.VMEM((4 * Hpt, E), jnp.float32),
            pltpu.VMEM((4 * Hpv, E), jnp.float32),
            pltpu.VMEM((4 * Hpa, E), jnp.float32),
        ],
        compiler_params=pltpu.CompilerParams(vmem_limit_bytes=_VMEM),
    )(*head_in)
    aux = {"sp_p_t": spt, "sp_p_v": spv, "sp_p_a": spa, "sp_s": sps}
    return o, aux


# speedup vs baseline: 1.1598x; 1.0028x over previous
import functools
import math

import jax
import jax.numpy as jnp
from jax.experimental import pallas as pl
from jax.experimental.pallas import tpu as pltpu

_EPS = 1e-5
_VMEM = 64 * 1024 * 1024


def _lstm_cell(z, c_prev, hp):
    i = jax.nn.sigmoid(z[:, 0 * hp:1 * hp])
    f = jax.nn.sigmoid(z[:, 1 * hp:2 * hp])
    g = jnp.tanh(z[:, 2 * hp:3 * hp])
    o = jax.nn.sigmoid(z[:, 3 * hp:4 * hp])
    c = f * c_prev + i * g
    return o * jnp.tanh(c), c


def _feat_kernel(*refs, T, B, dims):
    (xt, w1t, b1t, whh1t, w2t, b2t, lngt, lnbt, whh2t,
     xv, w1v, b1v, whh1v, w2v, b2v, lngv, lnbv, whh2v,
     xa, w1a, b1a, whh1a, w2a, b2a, lnga, lnba, whh2a,
     ut_ref, uv_ref, ua_ref,
     gt, gv, ga, h1t, h1v, h1a,
     w2tb, w2vp, w2ap, w1tb, w1vb, w1ab, p_scr, xtm, xvm, xam) = refs

    mods = []
    for (H, Hp), g_scr, h1, whh1, whh2, x, w1, b1, w2, b2, lng, \
            lnb, out in (
            (dims[0], gt, h1t, whh1t, whh2t, xt, w1t, b1t, w2t,
             b2t, lngt, lnbt, ut_ref),
            (dims[1], gv, h1v, whh1v, whh2v, xv, w1v, b1v, w2v,
             b2v, lngv, lnbv, uv_ref),
            (dims[2], ga, h1a, whh1a, whh2a, xa, w1a, b1a, w2a,
             b2a, lnga, lnba, ua_ref)):
        mods.append(dict(H=H, Hp=Hp, g=g_scr, h1=h1,
                         whh1=whh1, whh2=whh2, x=x, w1=w1, b1=b1, w2=w2,
                         b2=b2, lng=lng, lnb=lnb, out=out))

    bf = jnp.bfloat16
    w2scrs = (w2tb, w2vp, w2ap)
    for scr, m in zip(w2scrs, mods):
        H, Hp = m["H"], m["Hp"]
        N = scr.shape[1]
        scr[0:H, :] = m["w2"][0:H, :].astype(bf)
        scr[H:Hp, :] = jnp.zeros((Hp - H, N), bf)
        scr[Hp:Hp + H, :] = m["w2"][H:2 * H, :].astype(bf)
        scr[Hp + H:2 * Hp, :] = jnp.zeros((Hp - H, N), bf)
    w1scrs = (w1tb, w1vb, w1ab)
    for scr, m in zip(w1scrs, mods):
        scr[...] = m["w1"][...].astype(bf)

    TB = T * B
    ri = jax.lax.broadcasted_iota(jnp.int32, (TB, TB), 0)
    ci = jax.lax.broadcasted_iota(jnp.int32, (TB, TB), 1)
    p_scr[...] = jnp.where(ci == (ri % B) * T + ri // B, 1.0, 0.0)

    for m, xm, w1b in zip(mods, (xtm, xvm, xam), w1scrs):
        xm[...] = jnp.dot(p_scr[...], m["x"][...],
                          preferred_element_type=jnp.float32).astype(bf)
        m["g"][...] = (
            jnp.dot(xm[...], w1b[...],
                    preferred_element_type=jnp.float32) + m["b1"][...]
        )

    def run_layer(whh_key, store_h):
        st = [None] * len(mods)
        for t in range(T):
            zs = []
            for i, m in enumerate(mods):
                Hp, G4 = m["Hp"], 4 * m["Hp"]
                gf = m["g"][t * B:(t + 1) * B, 0:G4]
                gb = m["g"][(T - 1 - t) * B:(T - t) * B, G4:2 * G4]
                if t == 0:
                    zs.append((gf, gb))
                else:
                    w = m[whh_key]
                    zf = gf + jnp.dot(st[i][0], w[0:Hp, :],
                                      preferred_element_type=jnp.float32)
                    zb = gb + jnp.dot(st[i][1], w[Hp:2 * Hp, :],
                                      preferred_element_type=jnp.float32)
                    zs.append((zf, zb))
            for i, (m, (zf, zb)) in enumerate(zip(mods, zs)):
                Hp = m["Hp"]
                if t == 0:
                    i_f = jax.nn.sigmoid(zf[:, 0:Hp])
                    g_f = jnp.tanh(zf[:, 2 * Hp:3 * Hp])
                    o_f = jax.nn.sigmoid(zf[:, 3 * Hp:4 * Hp])
                    cf = i_f * g_f
                    hf = o_f * jnp.tanh(cf)
                    i_b = jax.nn.sigmoid(zb[:, 0:Hp])
                    g_b = jnp.tanh(zb[:, 2 * Hp:3 * Hp])
                    o_b = jax.nn.sigmoid(zb[:, 3 * Hp:4 * Hp])
                    cb = i_b * g_b
                    hb = o_b * jnp.tanh(cb)
                else:
                    hf, cf = _lstm_cell(zf, st[i][2], Hp)
                    hb, cb = _lstm_cell(zb, st[i][3], Hp)
                st[i] = (hf, hb, cf, cb)
                if store_h:
                    m["h1"][t * B:(t + 1) * B, 0:Hp] = hf
                    m["h1"][(T - 1 - t) * B:(T - t) * B, Hp:2 * Hp] = hb
        return st

    st1 = run_layer("whh1", store_h=True)
    for m, (hf, hb, _, _) in zip(mods, st1):
        Hp = m["Hp"]
        m["out"][:, 0 * Hp:1 * Hp] = hf
        m["out"][:, 2 * Hp:3 * Hp] = hb

    for scr, m in zip(w2scrs, mods):
        H, Hp = m["H"], m["Hp"]
        x1 = m["h1"][...]
        inv_n = 1.0 / (2 * H)
        mu = jnp.sum(x1, axis=1, keepdims=True) * inv_n
        ex2 = jnp.sum(x1 * x1, axis=1, keepdims=True) * inv_n
        xn = (x1 - mu) * jax.lax.rsqrt(ex2 - mu * mu + _EPS) * \
            m["lng"][...] + m["lnb"][...]
        m["g"][...] = (
            jnp.dot(xn.astype(bf), scr[...],
                    preferred_element_type=jnp.float32)
            + m["b2"][...]
        )

    st2 = run_layer("whh2", store_h=False)
    for m, (hf, hb, _, _) in zip(mods, st2):
        Hp = m["Hp"]
        m["out"][:, 1 * Hp:2 * Hp] = hf
        m["out"][:, 3 * Hp:4 * Hp] = hb


def _head_kernel(*refs, nhead, dims):
    (ut, uv, ua,
     pwt, pbt, pgt, ptt, pwv, pbv, pgv, ptv, pwa, pba, pga, pta,
     qtw, qtb, qvw, qvb, qaw, qab, shw, shb, sdw, sdb,
     inw, inb, ouw, oub, l1g, l1b, f1w, f1b, f2w, f2b, l2g, l2b,
     fw, fb, o_ref, st_ref, sv_ref, sa_ref, ss_ref,
     pts, pvs, pas) = refs

    E = shw.shape[0]
    B = ut.shape[0]
    S = 6
    SB = S * B
    dh = E // nhead
    scale = 1.0 / math.sqrt(dh)

    for scr, w, (H, Hp) in ((pts, pwt, dims[0]), (pvs, pwv, dims[1]),
                            (pas, pwa, dims[2])):
        if H == Hp:
            scr[...] = w[...]
        else:
            for k in range(4):
                scr[k * Hp:k * Hp + H, :] = w[k * H:(k + 1) * H, :]
                scr[k * Hp + H:(k + 1) * Hp, :] = jnp.zeros(
                    (Hp - H, E), jnp.float32)
    pwt, pwv, pwa = pts, pvs, pas

    def ln(x, g, b):
        mu = jnp.mean(x, axis=-1, keepdims=True)
        xc = x - mu
        var = jnp.mean(xc * xc, axis=-1, keepdims=True)
        return xc * jax.lax.rsqrt(var + _EPS) * g[...] + b[...]

    def lin(x, w, b):
        return jnp.dot(x, w[...], preferred_element_type=jnp.float32) + b[...]

    t = ln(jnp.maximum(lin(ut[...], pwt, pbt), 0.0), pgt, ptt)
    v = ln(jnp.maximum(lin(uv[...], pwv, pbv), 0.0), pgv, ptv)
    a = ln(jnp.maximum(lin(ua[...], pwa, pba), 0.0), pga, pta)

    p_t = jax.nn.sigmoid(lin(t, qtw, qtb))
    p_v = jax.nn.sigmoid(lin(v, qvw, qvb))
    p_a = jax.nn.sigmoid(lin(a, qaw, qab))
    s_t = jax.nn.sigmoid(lin(t, shw, shb))
    s_v = jax.nn.sigmoid(lin(v, shw, shb))
    s_a = jax.nn.sigmoid(lin(a, shw, shb))

    st_ref[...] = lin(p_t, sdw, sdb)
    sv_ref[...] = lin(p_v, sdw, sdb)
    sa_ref[...] = lin(p_a, sdw, sdb)
    ss_ref[...] = lin((s_t + s_v + s_a) / 3.0, sdw, sdb)

    h = jnp.concatenate([p_t, p_v, p_a, s_t, s_v, s_a], axis=0)

    qkv = lin(h, inw, inb)
    q, k, vv = qkv[:, :E], qkv[:, E:2 * E], qkv[:, 2 * E:]
    ri = jax.lax.broadcasted_iota(jnp.int32, (SB, SB), 0)
    rj = jax.lax.broadcasted_iota(jnp.int32, (SB, SB), 1)
    same = (ri % B) == (rj % B)

    attn = jnp.zeros((SB, E), jnp.float32)
    for hd in range(nhead):
        cs = slice(hd * dh, (hd + 1) * dh)
        sc = jax.lax.dot_general(
            q[:, cs], k[:, cs], dimension_numbers=(((1,), (1,)), ((), ())),
            preferred_element_type=jnp.float32) * scale
        sc = jnp.where(same, sc, -1e30)
        m = jnp.max(sc, axis=-1, keepdims=True)
        p = jnp.exp(sc - m)
        p = p / jnp.sum(p, axis=-1, keepdims=True)
        hv = jnp.dot(p, vv[:, cs], preferred_element_type=jnp.float32)
        attn = attn + jnp.dot(hv, ouw[cs, :],
                              preferred_element_type=jnp.float32)

    x = ln(h + attn + oub[...], l1g, l1b)
    x = ln(x + lin(jnp.maximum(lin(x, f1w, f1b), 0.0), f2w, f2b), l2g, l2b)

    o = jnp.zeros((B, fw.shape[1]), jnp.float32)
    for s in range(S):
        o = o + jnp.dot(x[s * B:(s + 1) * B, :], fw[s * E:(s + 1) * E, :],
                        preferred_element_type=jnp.float32)
    o_ref[...] = o + fb[...]


def _pad_vec(g, H, Hp):
    if H == Hp:
        return g.reshape(1, -1)
    z = jnp.zeros((Hp - H,), g.dtype)
    return jnp.concatenate([g[:H], z, g[H:], z]).reshape(1, -1)


def kernel(trnn1_w_ih, trnn1_b, trnn1_w_hh, trnn2_w_ih, trnn2_b, trnn2_w_hh,
           vrnn1_w_ih, vrnn1_b, vrnn1_w_hh, vrnn2_w_ih, vrnn2_b, vrnn2_w_hh,
           arnn1_w_ih, arnn1_b, arnn1_w_hh, arnn2_w_ih, arnn2_b, arnn2_w_hh,
           tln_g, tln_b, vln_g, vln_b, aln_g, aln_b,
           proj_t_w, proj_t_b, proj_t_ln_g, proj_t_ln_b,
           proj_v_w, proj_v_b, proj_v_ln_g, proj_v_ln_b,
           proj_a_w, proj_a_b, proj_a_ln_g, proj_a_ln_b,
           priv_t_w, priv_t_b, priv_v_w, priv_v_b, priv_a_w, priv_a_b,
           shared_w, shared_b, spd_w, spd_b, fusion_w, fusion_b,
           tx_in_w, tx_in_b, tx_out_w, tx_out_b,
           tx_ff1_w, tx_ff1_b, tx_ff2_w, tx_ff2_b,
           tx_ln1_g, tx_ln1_b, tx_ln2_g, tx_ln2_b,
           visual, acoustic, sentences):
    B, T, Ht = sentences.shape
    Hv = visual.shape[2]
    Ha = acoustic.shape[2]
    Hpt = trnn1_w_hh.shape[1]
    Hpv = vrnn1_w_hh.shape[1]
    Hpa = arnn1_w_hh.shape[1]

    xt = sentences.reshape(B * T, Ht)
    xv = visual.reshape(B * T, Hv)
    xa = acoustic.reshape(B * T, Ha)

    r = lambda z: z.reshape(1, -1)
    r2 = lambda z: z.reshape(-1, z.shape[-1])
    feat_in = (
        xt, trnn1_w_ih, r(trnn1_b), r2(trnn1_w_hh),
        trnn2_w_ih, r(trnn2_b),
        _pad_vec(tln_g, Ht, Hpt), _pad_vec(tln_b, Ht, Hpt), r2(trnn2_w_hh),
        xv, vrnn1_w_ih, r(vrnn1_b), r2(vrnn1_w_hh),
        vrnn2_w_ih, r(vrnn2_b),
        _pad_vec(vln_g, Hv, Hpv), _pad_vec(vln_b, Hv, Hpv), r2(vrnn2_w_hh),
        xa, arnn1_w_ih, r(arnn1_b), r2(arnn1_w_hh),
        arnn2_w_ih, r(arnn2_b),
        _pad_vec(aln_g, Ha, Hpa), _pad_vec(aln_b, Ha, Hpa), r2(arnn2_w_hh),
    )

    ut, uv, ua = pl.pallas_call(
        functools.partial(_feat_kernel, T=T, B=B,
                          dims=((Ht, Hpt), (Hv, Hpv), (Ha, Hpa))),
        out_shape=[
            jax.ShapeDtypeStruct((B, 4 * Hpt), jnp.float32),
            jax.ShapeDtypeStruct((B, 4 * Hpv), jnp.float32),
            jax.ShapeDtypeStruct((B, 4 * Hpa), jnp.float32),
        ],
        scratch_shapes=[
            pltpu.VMEM((T * B, 8 * Hpt), jnp.float32),
            pltpu.VMEM((T * B, 8 * Hpv), jnp.float32),
            pltpu.VMEM((T * B, 8 * Hpa), jnp.float32),
            pltpu.VMEM((T * B, 2 * Hpt), jnp.float32),
            pltpu.VMEM((T * B, 2 * Hpv), jnp.float32),
            pltpu.VMEM((T * B, 2 * Hpa), jnp.float32),
            pltpu.VMEM((2 * Hpt, 8 * Hpt), jnp.bfloat16),
            pltpu.VMEM((2 * Hpv, 8 * Hpv), jnp.bfloat16),
            pltpu.VMEM((2 * Hpa, 8 * Hpa), jnp.bfloat16),
            pltpu.VMEM((Ht, 8 * Hpt), jnp.bfloat16),
            pltpu.VMEM((Hv, 8 * Hpv), jnp.bfloat16),
            pltpu.VMEM((Ha, 8 * Hpa), jnp.bfloat16),
            pltpu.VMEM((T * B, T * B), jnp.float32),
            pltpu.VMEM((T * B, Ht), jnp.bfloat16),
            pltpu.VMEM((T * B, Hv), jnp.bfloat16),
            pltpu.VMEM((T * B, Ha), jnp.bfloat16),
        ],
        compiler_params=pltpu.CompilerParams(vmem_limit_bytes=_VMEM),
    )(*feat_in)

    head_in = (
        ut, uv, ua,
        proj_t_w, r(proj_t_b),
        r(proj_t_ln_g), r(proj_t_ln_b),
        proj_v_w, r(proj_v_b),
        r(proj_v_ln_g), r(proj_v_ln_b),
        proj_a_w, r(proj_a_b),
        r(proj_a_ln_g), r(proj_a_ln_b),
        priv_t_w, r(priv_t_b), priv_v_w, r(priv_v_b), priv_a_w, r(priv_a_b),
        shared_w, r(shared_b), spd_w, r(spd_b),
        tx_in_w, r(tx_in_b), tx_out_w, r(tx_out_b),
        r(tx_ln1_g), r(tx_ln1_b),
        tx_ff1_w, r(tx_ff1_b), tx_ff2_w, r(tx_ff2_b),
        r(tx_ln2_g), r(tx_ln2_b),
        fusion_w, r(fusion_b),
    )
    E = shared_w.shape[0]
    o, spt, spv, spa, sps = pl.pallas_call(
        functools.partial(_head_kernel, nhead=2,
                          dims=((Ht, Hpt), (Hv, Hpv), (Ha, Hpa))),
        out_shape=(
            jax.ShapeDtypeStruct((B, 3 * E), jnp.float32),
            jax.ShapeDtypeStruct((B, 4), jnp.float32),
            jax.ShapeDtypeStruct((B, 4), jnp.float32),
            jax.ShapeDtypeStruct((B, 4), jnp.float32),
            jax.ShapeDtypeStruct((B, 4), jnp.float32),
        ),
        scratch_shapes=[
            pltpu.VMEM((4 * Hpt, E), jnp.float32),
            pltpu.VMEM((4 * Hpv, E), jnp.float32),
            pltpu.VMEM((4 * Hpa, E), jnp.float32),
        ],
        compiler_params=pltpu.CompilerParams(vmem_limit_bytes=_VMEM),
    )(*head_in)
    aux = {"sp_p_t": spt, "sp_p_v": spv, "sp_p_a": spa, "sp_s": sps}
    return o, aux


# speedup vs baseline: 1.2251x; 1.0563x over previous
import functools
import math

import jax
import jax.numpy as jnp
from jax.experimental import pallas as pl
from jax.experimental.pallas import tpu as pltpu

_EPS = 1e-5
_VMEM = 64 * 1024 * 1024


def _lstm_cell(z, c_prev, hp):
    i = jax.nn.sigmoid(z[:, 0 * hp:1 * hp])
    f = jax.nn.sigmoid(z[:, 1 * hp:2 * hp])
    g = jnp.tanh(z[:, 2 * hp:3 * hp])
    o = jax.nn.sigmoid(z[:, 3 * hp:4 * hp])
    c = f * c_prev + i * g
    return o * jnp.tanh(c), c


def _feat_kernel(*refs, T, B, dims):
    (xt, w1t, b1t, whh1t, w2t, b2t, lngt, lnbt, whh2t,
     xv, w1v, b1v, whh1v, w2v, b2v, lngv, lnbv, whh2v,
     xa, w1a, b1a, whh1a, w2a, b2a, lnga, lnba, whh2a,
     ut_ref, uv_ref, ua_ref,
     gt, gv, ga, h1t, h1v, h1a,
     w2tb, w2vp, w2ap, w1tb, w1vb, w1ab, p_scr, xtm, xvm, xam,
     w2t_lm, whh2t_lm, sem1, sem2) = refs

    cp1 = pltpu.make_async_copy(w2t, w2t_lm, sem1)
    cp2 = pltpu.make_async_copy(whh2t, whh2t_lm, sem2)
    cp1.start()
    cp2.start()
    w2t = w2t_lm
    whh2t = whh2t_lm

    mods = []
    for (H, Hp), g_scr, h1, whh1, whh2, x, w1, b1, w2, b2, lng, \
            lnb, out in (
            (dims[0], gt, h1t, whh1t, whh2t, xt, w1t, b1t, w2t,
             b2t, lngt, lnbt, ut_ref),
            (dims[1], gv, h1v, whh1v, whh2v, xv, w1v, b1v, w2v,
             b2v, lngv, lnbv, uv_ref),
            (dims[2], ga, h1a, whh1a, whh2a, xa, w1a, b1a, w2a,
             b2a, lnga, lnba, ua_ref)):
        mods.append(dict(H=H, Hp=Hp, g=g_scr, h1=h1,
                         whh1=whh1, whh2=whh2, x=x, w1=w1, b1=b1, w2=w2,
                         b2=b2, lng=lng, lnb=lnb, out=out))

    bf = jnp.bfloat16
    w1scrs = (w1tb, w1vb, w1ab)
    for scr, m in zip(w1scrs, mods):
        scr[...] = m["w1"][...].astype(bf)

    def build_w2_pads():
        for scr, m in zip((w2tb, w2vp, w2ap), mods):
            H, Hp = m["H"], m["Hp"]
            N = scr.shape[1]
            scr[0:H, :] = m["w2"][0:H, :].astype(bf)
            scr[H:Hp, :] = jnp.zeros((Hp - H, N), bf)
            scr[Hp:Hp + H, :] = m["w2"][H:2 * H, :].astype(bf)
            scr[Hp + H:2 * Hp, :] = jnp.zeros((Hp - H, N), bf)

    TB = T * B
    ri = jax.lax.broadcasted_iota(jnp.int32, (TB, TB), 0)
    ci = jax.lax.broadcasted_iota(jnp.int32, (TB, TB), 1)
    p_scr[...] = jnp.where(ci == (ri % B) * T + ri // B, 1.0, 0.0)

    for m, xm, w1b in zip(mods, (xtm, xvm, xam), w1scrs):
        xm[...] = jnp.dot(p_scr[...], m["x"][...],
                          preferred_element_type=jnp.float32).astype(bf)
        m["g"][...] = (
            jnp.dot(xm[...], w1b[...],
                    preferred_element_type=jnp.float32) + m["b1"][...]
        )

    def run_layer(whh_key, store_h):
        st = [None] * len(mods)
        for t in range(T):
            zs = []
            for i, m in enumerate(mods):
                Hp, G4 = m["Hp"], 4 * m["Hp"]
                gf = m["g"][t * B:(t + 1) * B, 0:G4]
                gb = m["g"][(T - 1 - t) * B:(T - t) * B, G4:2 * G4]
                if t == 0:
                    zs.append((gf, gb))
                else:
                    w = m[whh_key]
                    zf = gf + jnp.dot(st[i][0], w[0:Hp, :],
                                      preferred_element_type=jnp.float32)
                    zb = gb + jnp.dot(st[i][1], w[Hp:2 * Hp, :],
                                      preferred_element_type=jnp.float32)
                    zs.append((zf, zb))
            for i, (m, (zf, zb)) in enumerate(zip(mods, zs)):
                Hp = m["Hp"]
                if t == 0:
                    i_f = jax.nn.sigmoid(zf[:, 0:Hp])
                    g_f = jnp.tanh(zf[:, 2 * Hp:3 * Hp])
                    o_f = jax.nn.sigmoid(zf[:, 3 * Hp:4 * Hp])
                    cf = i_f * g_f
                    hf = o_f * jnp.tanh(cf)
                    i_b = jax.nn.sigmoid(zb[:, 0:Hp])
                    g_b = jnp.tanh(zb[:, 2 * Hp:3 * Hp])
                    o_b = jax.nn.sigmoid(zb[:, 3 * Hp:4 * Hp])
                    cb = i_b * g_b
                    hb = o_b * jnp.tanh(cb)
                else:
                    hf, cf = _lstm_cell(zf, st[i][2], Hp)
                    hb, cb = _lstm_cell(zb, st[i][3], Hp)
                st[i] = (hf, hb, cf, cb)
                if store_h:
                    m["h1"][t * B:(t + 1) * B, 0:Hp] = hf
                    m["h1"][(T - 1 - t) * B:(T - t) * B, Hp:2 * Hp] = hb
        return st

    st1 = run_layer("whh1", store_h=True)
    for m, (hf, hb, _, _) in zip(mods, st1):
        Hp = m["Hp"]
        m["out"][:, 0 * Hp:1 * Hp] = hf
        m["out"][:, 2 * Hp:3 * Hp] = hb

    cp1.wait()
    cp2.wait()
    build_w2_pads()

    for scr, m in zip((w2tb, w2vp, w2ap), mods):
        H, Hp = m["H"], m["Hp"]
        x1 = m["h1"][...]
        inv_n = 1.0 / (2 * H)
        mu = jnp.sum(x1, axis=1, keepdims=True) * inv_n
        ex2 = jnp.sum(x1 * x1, axis=1, keepdims=True) * inv_n
        xn = (x1 - mu) * jax.lax.rsqrt(ex2 - mu * mu + _EPS) * \
            m["lng"][...] + m["lnb"][...]
        m["g"][...] = (
            jnp.dot(xn.astype(bf), scr[...],
                    preferred_element_type=jnp.float32)
            + m["b2"][...]
        )

    st2 = run_layer("whh2", store_h=False)
    for m, (hf, hb, _, _) in zip(mods, st2):
        Hp = m["Hp"]
        m["out"][:, 1 * Hp:2 * Hp] = hf
        m["out"][:, 3 * Hp:4 * Hp] = hb


def _head_kernel(*refs, nhead, dims):
    (ut, uv, ua,
     pwt, pbt, pgt, ptt, pwv, pbv, pgv, ptv, pwa, pba, pga, pta,
     qtw, qtb, qvw, qvb, qaw, qab, shw, shb, sdw, sdb,
     inw, inb, ouw, oub, l1g, l1b, f1w, f1b, f2w, f2b, l2g, l2b,
     fw, fb, o_ref, st_ref, sv_ref, sa_ref, ss_ref,
     pts, pvs, pas) = refs

    E = shw.shape[0]
    B = ut.shape[0]
    S = 6
    SB = S * B
    dh = E // nhead
    scale = 1.0 / math.sqrt(dh)

    for scr, w, (H, Hp) in ((pts, pwt, dims[0]), (pvs, pwv, dims[1]),
                            (pas, pwa, dims[2])):
        if H == Hp:
            scr[...] = w[...]
        else:
            for k in range(4):
                scr[k * Hp:k * Hp + H, :] = w[k * H:(k + 1) * H, :]
                scr[k * Hp + H:(k + 1) * Hp, :] = jnp.zeros(
                    (Hp - H, E), jnp.float32)
    pwt, pwv, pwa = pts, pvs, pas

    def ln(x, g, b):
        mu = jnp.mean(x, axis=-1, keepdims=True)
        xc = x - mu
        var = jnp.mean(xc * xc, axis=-1, keepdims=True)
        return xc * jax.lax.rsqrt(var + _EPS) * g[...] + b[...]

    def lin(x, w, b):
        return jnp.dot(x, w[...], preferred_element_type=jnp.float32) + b[...]

    t = ln(jnp.maximum(lin(ut[...], pwt, pbt), 0.0), pgt, ptt)
    v = ln(jnp.maximum(lin(uv[...], pwv, pbv), 0.0), pgv, ptv)
    a = ln(jnp.maximum(lin(ua[...], pwa, pba), 0.0), pga, pta)

    p_t = jax.nn.sigmoid(lin(t, qtw, qtb))
    p_v = jax.nn.sigmoid(lin(v, qvw, qvb))
    p_a = jax.nn.sigmoid(lin(a, qaw, qab))
    s_t = jax.nn.sigmoid(lin(t, shw, shb))
    s_v = jax.nn.sigmoid(lin(v, shw, shb))
    s_a = jax.nn.sigmoid(lin(a, shw, shb))

    st_ref[...] = lin(p_t, sdw, sdb)
    sv_ref[...] = lin(p_v, sdw, sdb)
    sa_ref[...] = lin(p_a, sdw, sdb)
    ss_ref[...] = lin((s_t + s_v + s_a) / 3.0, sdw, sdb)

    h = jnp.concatenate([p_t, p_v, p_a, s_t, s_v, s_a], axis=0)

    qkv = lin(h, inw, inb)
    q, k, vv = qkv[:, :E], qkv[:, E:2 * E], qkv[:, 2 * E:]
    ri = jax.lax.broadcasted_iota(jnp.int32, (SB, SB), 0)
    rj = jax.lax.broadcasted_iota(jnp.int32, (SB, SB), 1)
    same = (ri % B) == (rj % B)

    attn = jnp.zeros((SB, E), jnp.float32)
    for hd in range(nhead):
        cs = slice(hd * dh, (hd + 1) * dh)
        sc = jax.lax.dot_general(
            q[:, cs], k[:, cs], dimension_numbers=(((1,), (1,)), ((), ())),
            preferred_element_type=jnp.float32) * scale
        sc = jnp.where(same, sc, -1e30)
        m = jnp.max(sc, axis=-1, keepdims=True)
        p = jnp.exp(sc - m)
        p = p / jnp.sum(p, axis=-1, keepdims=True)
        hv = jnp.dot(p, vv[:, cs], preferred_element_type=jnp.float32)
        attn = attn + jnp.dot(hv, ouw[cs, :],
                              preferred_element_type=jnp.float32)

    x = ln(h + attn + oub[...], l1g, l1b)
    x = ln(x + lin(jnp.maximum(lin(x, f1w, f1b), 0.0), f2w, f2b), l2g, l2b)

    o = jnp.zeros((B, fw.shape[1]), jnp.float32)
    for s in range(S):
        o = o + jnp.dot(x[s * B:(s + 1) * B, :], fw[s * E:(s + 1) * E, :],
                        preferred_element_type=jnp.float32)
    o_ref[...] = o + fb[...]


def _pad_vec(g, H, Hp):
    if H == Hp:
        return g.reshape(1, -1)
    z = jnp.zeros((Hp - H,), g.dtype)
    return jnp.concatenate([g[:H], z, g[H:], z]).reshape(1, -1)


def kernel(trnn1_w_ih, trnn1_b, trnn1_w_hh, trnn2_w_ih, trnn2_b, trnn2_w_hh,
           vrnn1_w_ih, vrnn1_b, vrnn1_w_hh, vrnn2_w_ih, vrnn2_b, vrnn2_w_hh,
           arnn1_w_ih, arnn1_b, arnn1_w_hh, arnn2_w_ih, arnn2_b, arnn2_w_hh,
           tln_g, tln_b, vln_g, vln_b, aln_g, aln_b,
           proj_t_w, proj_t_b, proj_t_ln_g, proj_t_ln_b,
           proj_v_w, proj_v_b, proj_v_ln_g, proj_v_ln_b,
           proj_a_w, proj_a_b, proj_a_ln_g, proj_a_ln_b,
           priv_t_w, priv_t_b, priv_v_w, priv_v_b, priv_a_w, priv_a_b,
           shared_w, shared_b, spd_w, spd_b, fusion_w, fusion_b,
           tx_in_w, tx_in_b, tx_out_w, tx_out_b,
           tx_ff1_w, tx_ff1_b, tx_ff2_w, tx_ff2_b,
           tx_ln1_g, tx_ln1_b, tx_ln2_g, tx_ln2_b,
           visual, acoustic, sentences):
    B, T, Ht = sentences.shape
    Hv = visual.shape[2]
    Ha = acoustic.shape[2]
    Hpt = trnn1_w_hh.shape[1]
    Hpv = vrnn1_w_hh.shape[1]
    Hpa = arnn1_w_hh.shape[1]

    xt = sentences.reshape(B * T, Ht)
    xv = visual.reshape(B * T, Hv)
    xa = acoustic.reshape(B * T, Ha)

    r = lambda z: z.reshape(1, -1)
    r2 = lambda z: z.reshape(-1, z.shape[-1])
    feat_in = (
        xt, trnn1_w_ih, r(trnn1_b), r2(trnn1_w_hh),
        trnn2_w_ih, r(trnn2_b),
        _pad_vec(tln_g, Ht, Hpt), _pad_vec(tln_b, Ht, Hpt), r2(trnn2_w_hh),
        xv, vrnn1_w_ih, r(vrnn1_b), r2(vrnn1_w_hh),
        vrnn2_w_ih, r(vrnn2_b),
        _pad_vec(vln_g, Hv, Hpv), _pad_vec(vln_b, Hv, Hpv), r2(vrnn2_w_hh),
        xa, arnn1_w_ih, r(arnn1_b), r2(arnn1_w_hh),
        arnn2_w_ih, r(arnn2_b),
        _pad_vec(aln_g, Ha, Hpa), _pad_vec(aln_b, Ha, Hpa), r2(arnn2_w_hh),
    )

    any_idx = {4, 8}
    in_specs = [
        pl.BlockSpec(memory_space=pl.ANY) if i in any_idx
        else pl.BlockSpec(a.shape)
        for i, a in enumerate(feat_in)
    ]
    ut, uv, ua = pl.pallas_call(
        functools.partial(_feat_kernel, T=T, B=B,
                          dims=((Ht, Hpt), (Hv, Hpv), (Ha, Hpa))),
        in_specs=in_specs,
        out_shape=[
            jax.ShapeDtypeStruct((B, 4 * Hpt), jnp.float32),
            jax.ShapeDtypeStruct((B, 4 * Hpv), jnp.float32),
            jax.ShapeDtypeStruct((B, 4 * Hpa), jnp.float32),
        ],
        scratch_shapes=[
            pltpu.VMEM((T * B, 8 * Hpt), jnp.float32),
            pltpu.VMEM((T * B, 8 * Hpv), jnp.float32),
            pltpu.VMEM((T * B, 8 * Hpa), jnp.float32),
            pltpu.VMEM((T * B, 2 * Hpt), jnp.float32),
            pltpu.VMEM((T * B, 2 * Hpv), jnp.float32),
            pltpu.VMEM((T * B, 2 * Hpa), jnp.float32),
            pltpu.VMEM((2 * Hpt, 8 * Hpt), jnp.bfloat16),
            pltpu.VMEM((2 * Hpv, 8 * Hpv), jnp.bfloat16),
            pltpu.VMEM((2 * Hpa, 8 * Hpa), jnp.bfloat16),
            pltpu.VMEM((Ht, 8 * Hpt), jnp.bfloat16),
            pltpu.VMEM((Hv, 8 * Hpv), jnp.bfloat16),
            pltpu.VMEM((Ha, 8 * Hpa), jnp.bfloat16),
            pltpu.VMEM((T * B, T * B), jnp.float32),
            pltpu.VMEM((T * B, Ht), jnp.bfloat16),
            pltpu.VMEM((T * B, Hv), jnp.bfloat16),
            pltpu.VMEM((T * B, Ha), jnp.bfloat16),
            pltpu.VMEM((2 * Ht, 8 * Hpt), jnp.float32),
            pltpu.VMEM((2 * Hpt, 4 * Hpt), jnp.float32),
            pltpu.SemaphoreType.DMA,
            pltpu.SemaphoreType.DMA,
        ],
        compiler_params=pltpu.CompilerParams(vmem_limit_bytes=_VMEM),
    )(*feat_in)

    head_in = (
        ut, uv, ua,
        proj_t_w, r(proj_t_b),
        r(proj_t_ln_g), r(proj_t_ln_b),
        proj_v_w, r(proj_v_b),
        r(proj_v_ln_g), r(proj_v_ln_b),
        proj_a_w, r(proj_a_b),
        r(proj_a_ln_g), r(proj_a_ln_b),
        priv_t_w, r(priv_t_b), priv_v_w, r(priv_v_b), priv_a_w, r(priv_a_b),
        shared_w, r(shared_b), spd_w, r(spd_b),
        tx_in_w, r(tx_in_b), tx_out_w, r(tx_out_b),
        r(tx_ln1_g), r(tx_ln1_b),
        tx_ff1_w, r(tx_ff1_b), tx_ff2_w, r(tx_ff2_b),
        r(tx_ln2_g), r(tx_ln2_b),
        fusion_w, r(fusion_b),
    )
    E = shared_w.shape[0]
    o, spt, spv, spa, sps = pl.pallas_call(
        functools.partial(_head_kernel, nhead=2,
                          dims=((Ht, Hpt), (Hv, Hpv), (Ha, Hpa))),
        out_shape=(
            jax.ShapeDtypeStruct((B, 3 * E), jnp.float32),
            jax.ShapeDtypeStruct((B, 4), jnp.float32),
            jax.ShapeDtypeStruct((B, 4), jnp.float32),
            jax.ShapeDtypeStruct((B, 4), jnp.float32),
            jax.ShapeDtypeStruct((B, 4), jnp.float32),
        ),
        scratch_shapes=[
            pltpu.VMEM((4 * Hpt, E), jnp.float32),
            pltpu.VMEM((4 * Hpv, E), jnp.float32),
            pltpu.VMEM((4 * Hpa, E), jnp.float32),
        ],
        compiler_params=pltpu.CompilerParams(vmem_limit_bytes=_VMEM),
    )(*head_in)
    aux = {"sp_p_t": spt, "sp_p_v": spv, "sp_p_a": spa, "sp_s": sps}
    return o, aux


# speedup vs baseline: 1.2523x; 1.0223x over previous
import functools
import math

import jax
import jax.numpy as jnp
from jax.experimental import pallas as pl
from jax.experimental.pallas import tpu as pltpu

_EPS = 1e-5
_VMEM = 64 * 1024 * 1024


def _lstm_cell(z, c_prev, hp):
    i = jax.nn.sigmoid(z[:, 0 * hp:1 * hp])
    f = jax.nn.sigmoid(z[:, 1 * hp:2 * hp])
    g = jnp.tanh(z[:, 2 * hp:3 * hp])
    o = jax.nn.sigmoid(z[:, 3 * hp:4 * hp])
    c = f * c_prev + i * g
    return o * jnp.tanh(c), c


def _feat_kernel(*refs, T, B, dims):
    (xt, w1t, b1t, whh1t, w2t, b2t, lngt, lnbt, whh2t,
     xv, w1v, b1v, whh1v, w2v, b2v, lngv, lnbv, whh2v,
     xa, w1a, b1a, whh1a, w2a, b2a, lnga, lnba, whh2a,
     ut_ref, uv_ref, ua_ref,
     gt, gv, ga, h1t, h1v, h1a,
     w2tb, w2vp, w2ap, w1tb, w1vb, w1ab, p_scr, xtm, xvm, xam,
     w2t_lm, whh2t_lm, whh1t_lm, sem1, sem2, sem3) = refs

    cp1 = pltpu.make_async_copy(w2t, w2t_lm, sem1)
    cp2 = pltpu.make_async_copy(whh2t, whh2t_lm, sem2)
    cp3 = pltpu.make_async_copy(whh1t, whh1t_lm, sem3)
    cp1.start()
    cp2.start()
    cp3.start()
    w2t = w2t_lm
    whh2t = whh2t_lm
    whh1t = whh1t_lm

    mods = []
    for (H, Hp), g_scr, h1, whh1, whh2, x, w1, b1, w2, b2, lng, \
            lnb, out in (
            (dims[0], gt, h1t, whh1t, whh2t, xt, w1t, b1t, w2t,
             b2t, lngt, lnbt, ut_ref),
            (dims[1], gv, h1v, whh1v, whh2v, xv, w1v, b1v, w2v,
             b2v, lngv, lnbv, uv_ref),
            (dims[2], ga, h1a, whh1a, whh2a, xa, w1a, b1a, w2a,
             b2a, lnga, lnba, ua_ref)):
        mods.append(dict(H=H, Hp=Hp, g=g_scr, h1=h1,
                         whh1=whh1, whh2=whh2, x=x, w1=w1, b1=b1, w2=w2,
                         b2=b2, lng=lng, lnb=lnb, out=out))

    bf = jnp.bfloat16
    w1scrs = (w1tb, w1vb, w1ab)
    for scr, m in zip(w1scrs, mods):
        scr[...] = m["w1"][...].astype(bf)

    def build_w2_pads():
        for scr, m in zip((w2tb, w2vp, w2ap), mods):
            H, Hp = m["H"], m["Hp"]
            N = scr.shape[1]
            scr[0:H, :] = m["w2"][0:H, :].astype(bf)
            scr[H:Hp, :] = jnp.zeros((Hp - H, N), bf)
            scr[Hp:Hp + H, :] = m["w2"][H:2 * H, :].astype(bf)
            scr[Hp + H:2 * Hp, :] = jnp.zeros((Hp - H, N), bf)

    TB = T * B
    ri = jax.lax.broadcasted_iota(jnp.int32, (TB, TB), 0)
    ci = jax.lax.broadcasted_iota(jnp.int32, (TB, TB), 1)
    p_scr[...] = jnp.where(ci == (ri % B) * T + ri // B, 1.0, 0.0)

    for m, xm, w1b in zip(mods, (xtm, xvm, xam), w1scrs):
        xm[...] = jnp.dot(p_scr[...], m["x"][...],
                          preferred_element_type=jnp.float32).astype(bf)
        m["g"][...] = (
            jnp.dot(xm[...], w1b[...],
                    preferred_element_type=jnp.float32) + m["b1"][...]
        )

    def run_layer(whh_key, store_h):
        st = [None] * len(mods)
        for t in range(T):
            zs = []
            for i, m in enumerate(mods):
                Hp, G4 = m["Hp"], 4 * m["Hp"]
                gf = m["g"][t * B:(t + 1) * B, 0:G4]
                gb = m["g"][(T - 1 - t) * B:(T - t) * B, G4:2 * G4]
                if t == 0:
                    zs.append((gf, gb))
                else:
                    w = m[whh_key]
                    zf = gf + jnp.dot(st[i][0], w[0:Hp, :],
                                      preferred_element_type=jnp.float32)
                    zb = gb + jnp.dot(st[i][1], w[Hp:2 * Hp, :],
                                      preferred_element_type=jnp.float32)
                    zs.append((zf, zb))
            for i, (m, (zf, zb)) in enumerate(zip(mods, zs)):
                Hp = m["Hp"]
                if t == 0:
                    i_f = jax.nn.sigmoid(zf[:, 0:Hp])
                    g_f = jnp.tanh(zf[:, 2 * Hp:3 * Hp])
                    o_f = jax.nn.sigmoid(zf[:, 3 * Hp:4 * Hp])
                    cf = i_f * g_f
                    hf = o_f * jnp.tanh(cf)
                    i_b = jax.nn.sigmoid(zb[:, 0:Hp])
                    g_b = jnp.tanh(zb[:, 2 * Hp:3 * Hp])
                    o_b = jax.nn.sigmoid(zb[:, 3 * Hp:4 * Hp])
                    cb = i_b * g_b
                    hb = o_b * jnp.tanh(cb)
                else:
                    hf, cf = _lstm_cell(zf, st[i][2], Hp)
                    hb, cb = _lstm_cell(zb, st[i][3], Hp)
                st[i] = (hf, hb, cf, cb)
                if store_h:
                    m["h1"][t * B:(t + 1) * B, 0:Hp] = hf
                    m["h1"][(T - 1 - t) * B:(T - t) * B, Hp:2 * Hp] = hb
        return st

    cp3.wait()
    st1 = run_layer("whh1", store_h=True)
    for m, (hf, hb, _, _) in zip(mods, st1):
        Hp = m["Hp"]
        m["out"][:, 0 * Hp:1 * Hp] = hf
        m["out"][:, 2 * Hp:3 * Hp] = hb

    cp1.wait()
    cp2.wait()
    build_w2_pads()

    for scr, m in zip((w2tb, w2vp, w2ap), mods):
        H, Hp = m["H"], m["Hp"]
        x1 = m["h1"][...]
        inv_n = 1.0 / (2 * H)
        mu = jnp.sum(x1, axis=1, keepdims=True) * inv_n
        ex2 = jnp.sum(x1 * x1, axis=1, keepdims=True) * inv_n
        xn = (x1 - mu) * jax.lax.rsqrt(ex2 - mu * mu + _EPS) * \
            m["lng"][...] + m["lnb"][...]
        m["g"][...] = (
            jnp.dot(xn.astype(bf), scr[...],
                    preferred_element_type=jnp.float32)
            + m["b2"][...]
        )

    st2 = run_layer("whh2", store_h=False)
    for m, (hf, hb, _, _) in zip(mods, st2):
        Hp = m["Hp"]
        m["out"][:, 1 * Hp:2 * Hp] = hf
        m["out"][:, 3 * Hp:4 * Hp] = hb


def _head_kernel(*refs, nhead, dims):
    (ut, uv, ua,
     pwt, pbt, pgt, ptt, pwv, pbv, pgv, ptv, pwa, pba, pga, pta,
     qtw, qtb, qvw, qvb, qaw, qab, shw, shb, sdw, sdb,
     inw, inb, ouw, oub, l1g, l1b, f1w, f1b, f2w, f2b, l2g, l2b,
     fw, fb, o_ref, st_ref, sv_ref, sa_ref, ss_ref,
     pts, pvs, pas) = refs

    E = shw.shape[0]
    B = ut.shape[0]
    S = 6
    SB = S * B
    dh = E // nhead
    scale = 1.0 / math.sqrt(dh)

    for scr, w, (H, Hp) in ((pts, pwt, dims[0]), (pvs, pwv, dims[1]),
                            (pas, pwa, dims[2])):
        if H == Hp:
            scr[...] = w[...]
        else:
            for k in range(4):
                scr[k * Hp:k * Hp + H, :] = w[k * H:(k + 1) * H, :]
                scr[k * Hp + H:(k + 1) * Hp, :] = jnp.zeros(
                    (Hp - H, E), jnp.float32)
    pwt, pwv, pwa = pts, pvs, pas

    def ln(x, g, b):
        mu = jnp.mean(x, axis=-1, keepdims=True)
        xc = x - mu
        var = jnp.mean(xc * xc, axis=-1, keepdims=True)
        return xc * jax.lax.rsqrt(var + _EPS) * g[...] + b[...]

    def lin(x, w, b):
        return jnp.dot(x, w[...], preferred_element_type=jnp.float32) + b[...]

    t = ln(jnp.maximum(lin(ut[...], pwt, pbt), 0.0), pgt, ptt)
    v = ln(jnp.maximum(lin(uv[...], pwv, pbv), 0.0), pgv, ptv)
    a = ln(jnp.maximum(lin(ua[...], pwa, pba), 0.0), pga, pta)

    p_t = jax.nn.sigmoid(lin(t, qtw, qtb))
    p_v = jax.nn.sigmoid(lin(v, qvw, qvb))
    p_a = jax.nn.sigmoid(lin(a, qaw, qab))
    s_t = jax.nn.sigmoid(lin(t, shw, shb))
    s_v = jax.nn.sigmoid(lin(v, shw, shb))
    s_a = jax.nn.sigmoid(lin(a, shw, shb))

    st_ref[...] = lin(p_t, sdw, sdb)
    sv_ref[...] = lin(p_v, sdw, sdb)
    sa_ref[...] = lin(p_a, sdw, sdb)
    ss_ref[...] = lin((s_t + s_v + s_a) / 3.0, sdw, sdb)

    h = jnp.concatenate([p_t, p_v, p_a, s_t, s_v, s_a], axis=0)

    qkv = lin(h, inw, inb)
    q, k, vv = qkv[:, :E], qkv[:, E:2 * E], qkv[:, 2 * E:]
    ri = jax.lax.broadcasted_iota(jnp.int32, (SB, SB), 0)
    rj = jax.lax.broadcasted_iota(jnp.int32, (SB, SB), 1)
    same = (ri % B) == (rj % B)

    attn = jnp.zeros((SB, E), jnp.float32)
    for hd in range(nhead):
        cs = slice(hd * dh, (hd + 1) * dh)
        sc = jax.lax.dot_general(
            q[:, cs], k[:, cs], dimension_numbers=(((1,), (1,)), ((), ())),
            preferred_element_type=jnp.float32) * scale
        sc = jnp.where(same, sc, -1e30)
        m = jnp.max(sc, axis=-1, keepdims=True)
        p = jnp.exp(sc - m)
        p = p / jnp.sum(p, axis=-1, keepdims=True)
        hv = jnp.dot(p, vv[:, cs], preferred_element_type=jnp.float32)
        attn = attn + jnp.dot(hv, ouw[cs, :],
                              preferred_element_type=jnp.float32)

    x = ln(h + attn + oub[...], l1g, l1b)
    x = ln(x + lin(jnp.maximum(lin(x, f1w, f1b), 0.0), f2w, f2b), l2g, l2b)

    o = jnp.zeros((B, fw.shape[1]), jnp.float32)
    for s in range(S):
        o = o + jnp.dot(x[s * B:(s + 1) * B, :], fw[s * E:(s + 1) * E, :],
                        preferred_element_type=jnp.float32)
    o_ref[...] = o + fb[...]


def _pad_vec(g, H, Hp):
    if H == Hp:
        return g.reshape(1, -1)
    z = jnp.zeros((Hp - H,), g.dtype)
    return jnp.concatenate([g[:H], z, g[H:], z]).reshape(1, -1)


def kernel(trnn1_w_ih, trnn1_b, trnn1_w_hh, trnn2_w_ih, trnn2_b, trnn2_w_hh,
           vrnn1_w_ih, vrnn1_b, vrnn1_w_hh, vrnn2_w_ih, vrnn2_b, vrnn2_w_hh,
           arnn1_w_ih, arnn1_b, arnn1_w_hh, arnn2_w_ih, arnn2_b, arnn2_w_hh,
           tln_g, tln_b, vln_g, vln_b, aln_g, aln_b,
           proj_t_w, proj_t_b, proj_t_ln_g, proj_t_ln_b,
           proj_v_w, proj_v_b, proj_v_ln_g, proj_v_ln_b,
           proj_a_w, proj_a_b, proj_a_ln_g, proj_a_ln_b,
           priv_t_w, priv_t_b, priv_v_w, priv_v_b, priv_a_w, priv_a_b,
           shared_w, shared_b, spd_w, spd_b, fusion_w, fusion_b,
           tx_in_w, tx_in_b, tx_out_w, tx_out_b,
           tx_ff1_w, tx_ff1_b, tx_ff2_w, tx_ff2_b,
           tx_ln1_g, tx_ln1_b, tx_ln2_g, tx_ln2_b,
           visual, acoustic, sentences):
    B, T, Ht = sentences.shape
    Hv = visual.shape[2]
    Ha = acoustic.shape[2]
    Hpt = trnn1_w_hh.shape[1]
    Hpv = vrnn1_w_hh.shape[1]
    Hpa = arnn1_w_hh.shape[1]

    xt = sentences.reshape(B * T, Ht)
    xv = visual.reshape(B * T, Hv)
    xa = acoustic.reshape(B * T, Ha)

    r = lambda z: z.reshape(1, -1)
    r2 = lambda z: z.reshape(-1, z.shape[-1])
    feat_in = (
        xt, trnn1_w_ih, r(trnn1_b), r2(trnn1_w_hh),
        trnn2_w_ih, r(trnn2_b),
        _pad_vec(tln_g, Ht, Hpt), _pad_vec(tln_b, Ht, Hpt), r2(trnn2_w_hh),
        xv, vrnn1_w_ih, r(vrnn1_b), r2(vrnn1_w_hh),
        vrnn2_w_ih, r(vrnn2_b),
        _pad_vec(vln_g, Hv, Hpv), _pad_vec(vln_b, Hv, Hpv), r2(vrnn2_w_hh),
        xa, arnn1_w_ih, r(arnn1_b), r2(arnn1_w_hh),
        arnn2_w_ih, r(arnn2_b),
        _pad_vec(aln_g, Ha, Hpa), _pad_vec(aln_b, Ha, Hpa), r2(arnn2_w_hh),
    )

    any_idx = {3, 4, 8}
    in_specs = [
        pl.BlockSpec(memory_space=pl.ANY) if i in any_idx
        else pl.BlockSpec(a.shape)
        for i, a in enumerate(feat_in)
    ]
    ut, uv, ua = pl.pallas_call(
        functools.partial(_feat_kernel, T=T, B=B,
                          dims=((Ht, Hpt), (Hv, Hpv), (Ha, Hpa))),
        in_specs=in_specs,
        out_shape=[
            jax.ShapeDtypeStruct((B, 4 * Hpt), jnp.float32),
            jax.ShapeDtypeStruct((B, 4 * Hpv), jnp.float32),
            jax.ShapeDtypeStruct((B, 4 * Hpa), jnp.float32),
        ],
        scratch_shapes=[
            pltpu.VMEM((T * B, 8 * Hpt), jnp.float32),
            pltpu.VMEM((T * B, 8 * Hpv), jnp.float32),
            pltpu.VMEM((T * B, 8 * Hpa), jnp.float32),
            pltpu.VMEM((T * B, 2 * Hpt), jnp.float32),
            pltpu.VMEM((T * B, 2 * Hpv), jnp.float32),
            pltpu.VMEM((T * B, 2 * Hpa), jnp.float32),
            pltpu.VMEM((2 * Hpt, 8 * Hpt), jnp.bfloat16),
            pltpu.VMEM((2 * Hpv, 8 * Hpv), jnp.bfloat16),
            pltpu.VMEM((2 * Hpa, 8 * Hpa), jnp.bfloat16),
            pltpu.VMEM((Ht, 8 * Hpt), jnp.bfloat16),
            pltpu.VMEM((Hv, 8 * Hpv), jnp.bfloat16),
            pltpu.VMEM((Ha, 8 * Hpa), jnp.bfloat16),
            pltpu.VMEM((T * B, T * B), jnp.float32),
            pltpu.VMEM((T * B, Ht), jnp.bfloat16),
            pltpu.VMEM((T * B, Hv), jnp.bfloat16),
            pltpu.VMEM((T * B, Ha), jnp.bfloat16),
            pltpu.VMEM((2 * Ht, 8 * Hpt), jnp.float32),
            pltpu.VMEM((2 * Hpt, 4 * Hpt), jnp.float32),
            pltpu.VMEM((2 * Hpt, 4 * Hpt), jnp.float32),
            pltpu.SemaphoreType.DMA,
            pltpu.SemaphoreType.DMA,
            pltpu.SemaphoreType.DMA,
        ],
        compiler_params=pltpu.CompilerParams(vmem_limit_bytes=_VMEM),
    )(*feat_in)

    head_in = (
        ut, uv, ua,
        proj_t_w, r(proj_t_b),
        r(proj_t_ln_g), r(proj_t_ln_b),
        proj_v_w, r(proj_v_b),
        r(proj_v_ln_g), r(proj_v_ln_b),
        proj_a_w, r(proj_a_b),
        r(proj_a_ln_g), r(proj_a_ln_b),
        priv_t_w, r(priv_t_b), priv_v_w, r(priv_v_b), priv_a_w, r(priv_a_b),
        shared_w, r(shared_b), spd_w, r(spd_b),
        tx_in_w, r(tx_in_b), tx_out_w, r(tx_out_b),
        r(tx_ln1_g), r(tx_ln1_b),
        tx_ff1_w, r(tx_ff1_b), tx_ff2_w, r(tx_ff2_b),
        r(tx_ln2_g), r(tx_ln2_b),
        fusion_w, r(fusion_b),
    )
    E = shared_w.shape[0]
    o, spt, spv, spa, sps = pl.pallas_call(
        functools.partial(_head_kernel, nhead=2,
                          dims=((Ht, Hpt), (Hv, Hpv), (Ha, Hpa))),
        out_shape=(
            jax.ShapeDtypeStruct((B, 3 * E), jnp.float32),
            jax.ShapeDtypeStruct((B, 4), jnp.float32),
            jax.ShapeDtypeStruct((B, 4), jnp.float32),
            jax.ShapeDtypeStruct((B, 4), jnp.float32),
            jax.ShapeDtypeStruct((B, 4), jnp.float32),
        ),
        scratch_shapes=[
            pltpu.VMEM((4 * Hpt, E), jnp.float32),
            pltpu.VMEM((4 * Hpv, E), jnp.float32),
            pltpu.VMEM((4 * Hpa, E), jnp.float32),
        ],
        compiler_params=pltpu.CompilerParams(vmem_limit_bytes=_VMEM),
    )(*head_in)
    aux = {"sp_p_t": spt, "sp_p_v": spv, "sp_p_a": spa, "sp_s": sps}
    return o, aux
